# Initial kernel scaffold; baseline (speedup 1.0000x reference)
import jax
import jax.numpy as jnp
from jax.experimental import pallas as pl


def kernel(x, edge_index, W1, b1, W2, b2, l0f_Wih, l0f_Whh, l0f_bih, l0f_bhh, l0b_Wih, l0b_Whh, l0b_bih, l0b_bhh, l1f_Wih, l1f_Whh, l1f_bih, l1f_bhh, l1b_Wih, l1b_Whh, l1b_bih, l1b_bhh, Wl, bl):
    raise NotImplementedError("write your pallas kernel here")


# speedup vs baseline: 10.2599x; 10.2599x over previous
import functools

import jax
import jax.numpy as jnp
from jax import lax
from jax.experimental import pallas as pl
from jax.experimental.pallas import tpu as pltpu
import jax.experimental.pallas.tpu_sc as plsc

N = 10000
E = 320000
NP = 10240
SROWS = NP // 16
CW = 80
NWORK = 32
KCH = 128
EPAD = NWORK * KCH * CW
GH = 64

_f32 = jnp.float32


def _mesh():
    return plsc.VectorSubcoreMesh(core_axis_name="c", subcore_axis_name="s")


def _sc_degree(dst2):
    @functools.partial(
        pl.kernel,
        out_type=jax.ShapeDtypeStruct((2, NP, 16), _f32),
        mesh=_mesh(),
        compiler_params=pltpu.CompilerParams(use_tc_tiling_on_sc=False),
        scratch_types=[
            pltpu.VMEM((KCH, CW), jnp.int32),
            pltpu.VMEM((CW, 16), _f32),
            pltpu.VMEM_SHARED((NP, 16), _f32),
        ],
    )
    def k(dst_hbm, ones_hbm, zeros_hbm, out_hbm, dst_v, ones_v, acc_sh):
        c = lax.axis_index("c")
        s = lax.axis_index("s")
        w = s * 2 + c
        pltpu.sync_copy(zeros_hbm.at[pl.ds(s * SROWS, SROWS)],
                        acc_sh.at[pl.ds(s * SROWS, SROWS)])
        pltpu.sync_copy(ones_hbm, ones_v)
        pltpu.sync_copy(dst_hbm.at[pl.ds(w * KCH, KCH)], dst_v)
        plsc.subcore_barrier()

        def body(j, carry):
            pltpu.sync_copy(ones_v, acc_sh.at[dst_v.at[j]], add=True)
            return carry

        lax.fori_loop(0, KCH, body, 0)
        plsc.subcore_barrier()
        pltpu.sync_copy(acc_sh.at[pl.ds(s * SROWS, SROWS)],
                        out_hbm.at[c, pl.ds(s * SROWS, SROWS)])

    return k(dst2, jnp.ones((CW, 16), _f32), jnp.zeros((NP, 16), _f32))


def _sc_scatter(hp, src2, dst2, width):
    @functools.partial(
        pl.kernel,
        out_type=jax.ShapeDtypeStruct((2, NP, width), _f32),
        mesh=_mesh(),
        compiler_params=pltpu.CompilerParams(use_tc_tiling_on_sc=False),
        scratch_types=[
            pltpu.VMEM((KCH, CW), jnp.int32),
            pltpu.VMEM((KCH, CW), jnp.int32),
            pltpu.VMEM((CW, width), _f32),
            pltpu.VMEM_SHARED((NP, width), _f32),
            pltpu.SemaphoreType.DMA,
        ],
    )
    def k(hp_hbm, src_hbm, dst_hbm, zeros_hbm, out_hbm,
          src_v, dst_v, rows_v, acc_sh, sem):
        c = lax.axis_index("c")
        s = lax.axis_index("s")
        w = s * 2 + c
        pltpu.sync_copy(zeros_hbm.at[pl.ds(s * SROWS, SROWS)],
                        acc_sh.at[pl.ds(s * SROWS, SROWS)])
        pltpu.sync_copy(src_hbm.at[pl.ds(w * KCH, KCH)], src_v)
        pltpu.sync_copy(dst_hbm.at[pl.ds(w * KCH, KCH)], dst_v)
        plsc.subcore_barrier()

        def body(j, carry):
            pltpu.async_copy(hp_hbm.at[src_v.at[j]], rows_v, sem).wait()
            pltpu.sync_copy(rows_v, acc_sh.at[dst_v.at[j]], add=True)
            return carry

        lax.fori_loop(0, KCH, body, 0)
        plsc.subcore_barrier()
        pltpu.sync_copy(acc_sh.at[pl.ds(s * SROWS, SROWS)],
                        out_hbm.at[c, pl.ds(s * SROWS, SROWS)])

    return k(hp, src2, dst2, jnp.zeros((NP, width), _f32))


_R = 1000
_G = N // _R


def _row_spec(width):
    return pl.BlockSpec((_R, width), lambda i: (i, 0))


def _full_spec(a, b):
    return pl.BlockSpec((a, b), lambda i: (0, 0))


def _prep1(dega, degb, x, W1):
    def body(da, db, xr, w1, h1p, dv):
        dinv = lax.rsqrt(da[...] + db[...] + 1.0)
        h1p[...] = jnp.dot(xr[...], w1[...],
                           preferred_element_type=_f32) * dinv
        dv[...] = dinv

    return pl.pallas_call(
        body,
        grid=(_G,),
        in_specs=[_row_spec(1), _row_spec(1), _row_spec(128),
                  _full_spec(128, 16)],
        out_specs=[_row_spec(16), _row_spec(1)],
        out_shape=[jax.ShapeDtypeStruct((N, 16), _f32),
                   jax.ShapeDtypeStruct((N, 1), _f32)],
    )(dega, degb, x, W1)


def _mid(a0, a1, h1p, dinv, b1r, W2):
    def body(a0r, a1r, hpr, dvr, b1ref, w2, out):
        h1 = jnp.maximum((a0r[...] + a1r[...] + hpr[...]) * dvr[...]
                         + b1ref[...], 0.0)
        out[...] = jnp.dot(h1, w2[...], preferred_element_type=_f32) * dvr[...]

    return pl.pallas_call(
        body,
        grid=(_G,),
        in_specs=[_row_spec(16), _row_spec(16), _row_spec(16), _row_spec(1),
                  _full_spec(1, 16), _full_spec(16, 32)],
        out_specs=_row_spec(32),
        out_shape=jax.ShapeDtypeStruct((N, 32), _f32),
    )(a0, a1, h1p, dinv, b1r, W2)


def _gi0(a0, a1, h2p, dinv, b2r, w0f, c0f, w0b, c0b):
    def body(a0r, a1r, hpr, dvr, b2ref, wf, cf, wb, cb, of, ob):
        h2 = jnp.maximum((a0r[...] + a1r[...] + hpr[...]) * dvr[...]
                         + b2ref[...], 0.0)
        of[...] = jnp.dot(h2, wf[...], preferred_element_type=_f32) + cf[...]
        ob[...] = jnp.dot(h2, wb[...], preferred_element_type=_f32) + cb[...]

    return pl.pallas_call(
        body,
        grid=(_G,),
        in_specs=[_row_spec(32), _row_spec(32), _row_spec(32), _row_spec(1),
                  _full_spec(1, 32), _full_spec(32, 384), _full_spec(1, 384),
                  _full_spec(32, 384), _full_spec(1, 384)],
        out_specs=[_row_spec(384), _row_spec(384)],
        out_shape=[jax.ShapeDtypeStruct((N, 384), _f32),
                   jax.ShapeDtypeStruct((N, 384), _f32)],
    )(a0, a1, h2p, dinv, b2r, w0f, c0f, w0b, c0b)


def _gi1(f0, r0, wft, wfb, c1f, wbt, wbb, c1b):
    def body(fr, rr, wft_, wfb_, cf, wbt_, wbb_, cb, of, ob):
        f = fr[...]
        r = rr[...]
        of[...] = (jnp.dot(f, wft_[...], preferred_element_type=_f32)
                   + jnp.dot(r, wfb_[...], preferred_element_type=_f32)
                   + cf[...])
        ob[...] = (jnp.dot(f, wbt_[...], preferred_element_type=_f32)
                   + jnp.dot(r, wbb_[...], preferred_element_type=_f32)
                   + cb[...])

    return pl.pallas_call(
        body,
        grid=(_G,),
        in_specs=[_row_spec(64), _row_spec(64),
                  _full_spec(64, 384), _full_spec(64, 384), _full_spec(1, 384),
                  _full_spec(64, 384), _full_spec(64, 384), _full_spec(1, 384)],
        out_specs=[_row_spec(384), _row_spec(384)],
        out_shape=[jax.ShapeDtypeStruct((N, 384), _f32),
                   jax.ShapeDtypeStruct((N, 384), _f32)],
    )(f0, r0, wft, wfb, c1f, wbt, wbb, c1b)


def _gru(gif, gib, wblk, bn):
    def body(gif_ref, gib_ref, wblk_ref, bn_ref, f_ref, r_ref):
        w = wblk_ref[...]
        bnv = bn_ref[...]

        def step(t, h):
            g = gif_ref[pl.ds(t, 1), :] + gib_ref[pl.ds(N - 1 - t, 1), :]
            gh = jnp.dot(h, w, preferred_element_type=_f32)
            rg = jax.nn.sigmoid(g[:, 0:128] + gh[:, 0:128])
            zg = jax.nn.sigmoid(g[:, 128:256] + gh[:, 128:256])
            ng = jnp.tanh(g[:, 256:384] + rg * (gh[:, 256:384] + bnv))
            h2 = (1.0 - zg) * ng + zg * h
            f_ref[pl.ds(t, 1), :] = h2[:, 0:64]
            r_ref[pl.ds(N - 1 - t, 1), :] = h2[:, 64:128]
            return h2

        lax.fori_loop(0, N, step, jnp.zeros((1, 128), _f32))

    return pl.pallas_call(
        body,
        out_shape=[jax.ShapeDtypeStruct((N, 64), _f32),
                   jax.ShapeDtypeStruct((N, 64), _f32)],
    )(gif, gib, wblk, bn)


def _final(f1, r1, wlt, wlb, blr):
    def body(fr, rr, wt, wb, bref, out):
        out[...] = (jnp.dot(fr[...], wt[...], preferred_element_type=_f32)
                    + jnp.dot(rr[...], wb[...], preferred_element_type=_f32)
                    + bref[...])

    return pl.pallas_call(
        body,
        grid=(_G,),
        in_specs=[_row_spec(64), _row_spec(64),
                  _full_spec(64, 10), _full_spec(64, 10), _full_spec(1, 10)],
        out_specs=_row_spec(10),
        out_shape=jax.ShapeDtypeStruct((N, 10), _f32),
    )(f1, r1, wlt, wlb, blr)


def _pack_gi(Wih, bih, bhh, off):
    T = Wih.T.astype(_f32)
    inn = T.shape[0]
    w = jnp.zeros((inn, 384), _f32)
    w = w.at[:, off:off + 64].set(T[:, 0:64])
    w = w.at[:, 128 + off:128 + off + 64].set(T[:, 64:128])
    w = w.at[:, 256 + off:256 + off + 64].set(T[:, 128:192])
    cv = jnp.zeros((384,), _f32)
    cv = cv.at[off:off + 64].set(bih[0:64] + bhh[0:64])
    cv = cv.at[128 + off:128 + off + 64].set(bih[64:128] + bhh[64:128])
    cv = cv.at[256 + off:256 + off + 64].set(bih[128:192])
    return w, cv.reshape(1, 384)


def _pack_whh(Whh_f, Whh_b, bhh_f, bhh_b):
    w = jnp.zeros((128, 384), _f32)
    w = w.at[0:64, 0:64].set(Whh_f[0:64].T)
    w = w.at[64:128, 64:128].set(Whh_b[0:64].T)
    w = w.at[0:64, 128:192].set(Whh_f[64:128].T)
    w = w.at[64:128, 192:256].set(Whh_b[64:128].T)
    w = w.at[0:64, 256:320].set(Whh_f[128:192].T)
    w = w.at[64:128, 320:384].set(Whh_b[128:192].T)
    bn = jnp.concatenate([bhh_f[128:192], bhh_b[128:192]]).reshape(1, 128)
    return w, bn


def kernel(x, edge_index, W1, b1, W2, b2,
           l0f_Wih, l0f_Whh, l0f_bih, l0f_bhh,
           l0b_Wih, l0b_Whh, l0b_bih, l0b_bhh,
           l1f_Wih, l1f_Whh, l1f_bih, l1f_bhh,
           l1b_Wih, l1b_Whh, l1b_bih, l1b_bhh, Wl, bl):
    pad = EPAD - E
    src2 = jnp.concatenate(
        [edge_index[0], jnp.zeros((pad,), jnp.int32)]).reshape(EPAD // CW, CW)
    dst2 = jnp.concatenate(
        [edge_index[1], jnp.full((pad,), N, jnp.int32)]).reshape(EPAD // CW, CW)

    degp = _sc_degree(dst2)
    h1p, dinv = _prep1(degp[0, :N, 0:1], degp[1, :N, 0:1], x, W1)
    acc1 = _sc_scatter(h1p, src2, dst2, 16)
    h2p = _mid(acc1[0, :N], acc1[1, :N], h1p, dinv, b1.reshape(1, 16), W2)
    acc2 = _sc_scatter(h2p, src2, dst2, 32)

    w0f, c0f = _pack_gi(l0f_Wih, l0f_bih, l0f_bhh, 0)
    w0b, c0b = _pack_gi(l0b_Wih, l0b_bih, l0b_bhh, 64)
    gi0f, gi0b = _gi0(acc2[0, :N], acc2[1, :N], h2p, dinv,
                      b2.reshape(1, 32), w0f, c0f, w0b, c0b)
    wblk0, bn0 = _pack_whh(l0f_Whh, l0b_Whh, l0f_bhh, l0b_bhh)
    f0, r0 = _gru(gi0f, gi0b, wblk0, bn0)

    w1f, c1f = _pack_gi(l1f_Wih, l1f_bih, l1f_bhh, 0)
    w1b, c1b = _pack_gi(l1b_Wih, l1b_bih, l1b_bhh, 64)
    gi1f, gi1b = _gi1(f0, r0, w1f[0:64], w1f[64:128], c1f,
                      w1b[0:64], w1b[64:128], c1b)
    wblk1, bn1 = _pack_whh(l1f_Whh, l1b_Whh, l1f_bhh, l1b_bhh)
    f1, r1 = _gru(gi1f, gi1b, wblk1, bn1)

    return _final(f1, r1, Wl[0:64], Wl[64:128], bl.reshape(1, 10))


# speedup vs baseline: 10.3757x; 1.0113x over previous
import functools

import jax
import jax.numpy as jnp
from jax import lax
from jax.experimental import pallas as pl
from jax.experimental.pallas import tpu as pltpu
import jax.experimental.pallas.tpu_sc as plsc

N = 10000
E = 320000
NP = 10240
SROWS = NP // 16
CW = 80
NWORK = 32
KCH = 128
EPAD = NWORK * KCH * CW
GH = 64

_f32 = jnp.float32


def _mesh():
    return plsc.VectorSubcoreMesh(core_axis_name="c", subcore_axis_name="s")


def _sc_degree(dst2):
    @functools.partial(
        pl.kernel,
        out_type=jax.ShapeDtypeStruct((2, NP, 16), _f32),
        mesh=_mesh(),
        compiler_params=pltpu.CompilerParams(use_tc_tiling_on_sc=False),
        scratch_types=[
            pltpu.VMEM((KCH, CW), jnp.int32),
            pltpu.VMEM((CW, 16), _f32),
            pltpu.VMEM_SHARED((NP, 16), _f32),
        ],
    )
    def k(dst_hbm, ones_hbm, zeros_hbm, out_hbm, dst_v, ones_v, acc_sh):
        c = lax.axis_index("c")
        s = lax.axis_index("s")
        w = s * 2 + c
        pltpu.sync_copy(zeros_hbm.at[pl.ds(s * SROWS, SROWS)],
                        acc_sh.at[pl.ds(s * SROWS, SROWS)])
        pltpu.sync_copy(ones_hbm, ones_v)
        pltpu.sync_copy(dst_hbm.at[pl.ds(w * KCH, KCH)], dst_v)
        plsc.subcore_barrier()

        def body(j, carry):
            pltpu.sync_copy(ones_v, acc_sh.at[dst_v.at[j]], add=True)
            return carry

        lax.fori_loop(0, KCH, body, 0)
        plsc.subcore_barrier()
        pltpu.sync_copy(acc_sh.at[pl.ds(s * SROWS, SROWS)],
                        out_hbm.at[c, pl.ds(s * SROWS, SROWS)])

    return k(dst2, jnp.ones((CW, 16), _f32), jnp.zeros((NP, 16), _f32))


def _sc_scatter(hp, src2, dst2, width):
    @functools.partial(
        pl.kernel,
        out_type=jax.ShapeDtypeStruct((2, NP, width), _f32),
        mesh=_mesh(),
        compiler_params=pltpu.CompilerParams(use_tc_tiling_on_sc=False),
        scratch_types=[
            pltpu.VMEM((KCH, CW), jnp.int32),
            pltpu.VMEM((KCH, CW), jnp.int32),
            pltpu.VMEM((CW, width), _f32),
            pltpu.VMEM_SHARED((NP, width), _f32),
            pltpu.SemaphoreType.DMA,
        ],
    )
    def k(hp_hbm, src_hbm, dst_hbm, zeros_hbm, out_hbm,
          src_v, dst_v, rows_v, acc_sh, sem):
        c = lax.axis_index("c")
        s = lax.axis_index("s")
        w = s * 2 + c
        pltpu.sync_copy(zeros_hbm.at[pl.ds(s * SROWS, SROWS)],
                        acc_sh.at[pl.ds(s * SROWS, SROWS)])
        pltpu.sync_copy(src_hbm.at[pl.ds(w * KCH, KCH)], src_v)
        pltpu.sync_copy(dst_hbm.at[pl.ds(w * KCH, KCH)], dst_v)
        plsc.subcore_barrier()

        def body(j, carry):
            pltpu.async_copy(hp_hbm.at[src_v.at[j]], rows_v, sem).wait()
            pltpu.sync_copy(rows_v, acc_sh.at[dst_v.at[j]], add=True)
            return carry

        lax.fori_loop(0, KCH, body, 0)
        plsc.subcore_barrier()
        pltpu.sync_copy(acc_sh.at[pl.ds(s * SROWS, SROWS)],
                        out_hbm.at[c, pl.ds(s * SROWS, SROWS)])

    return k(hp, src2, dst2, jnp.zeros((NP, width), _f32))


_R = 1000
_G = N // _R


def _row_spec(width):
    return pl.BlockSpec((_R, width), lambda i: (i, 0))


def _full_spec(a, b):
    return pl.BlockSpec((a, b), lambda i: (0, 0))


def _prep1(dega, degb, x, W1):
    def body(da, db, xr, w1, h1p, dv):
        dinv = lax.rsqrt(da[...] + db[...] + 1.0)
        h1p[...] = jnp.dot(xr[...], w1[...],
                           preferred_element_type=_f32) * dinv
        dv[...] = dinv

    return pl.pallas_call(
        body,
        grid=(_G,),
        in_specs=[_row_spec(1), _row_spec(1), _row_spec(128),
                  _full_spec(128, 16)],
        out_specs=[_row_spec(16), _row_spec(1)],
        out_shape=[jax.ShapeDtypeStruct((N, 16), _f32),
                   jax.ShapeDtypeStruct((N, 1), _f32)],
    )(dega, degb, x, W1)


def _mid(a0, a1, h1p, dinv, b1r, W2):
    def body(a0r, a1r, hpr, dvr, b1ref, w2, out):
        h1 = jnp.maximum((a0r[...] + a1r[...] + hpr[...]) * dvr[...]
                         + b1ref[...], 0.0)
        out[...] = jnp.dot(h1, w2[...], preferred_element_type=_f32) * dvr[...]

    return pl.pallas_call(
        body,
        grid=(_G,),
        in_specs=[_row_spec(16), _row_spec(16), _row_spec(16), _row_spec(1),
                  _full_spec(1, 16), _full_spec(16, 32)],
        out_specs=_row_spec(32),
        out_shape=jax.ShapeDtypeStruct((N, 32), _f32),
    )(a0, a1, h1p, dinv, b1r, W2)


def _gi0(a0, a1, h2p, dinv, b2r, w0f, c0f, w0b, c0b):
    def body(a0r, a1r, hpr, dvr, b2ref, wf, cf, wb, cb, of, ob):
        h2 = jnp.maximum((a0r[...] + a1r[...] + hpr[...]) * dvr[...]
                         + b2ref[...], 0.0)
        of[...] = jnp.dot(h2, wf[...], preferred_element_type=_f32) + cf[...]
        ob[...] = jnp.dot(h2, wb[...], preferred_element_type=_f32) + cb[...]

    return pl.pallas_call(
        body,
        grid=(_G,),
        in_specs=[_row_spec(32), _row_spec(32), _row_spec(32), _row_spec(1),
                  _full_spec(1, 32), _full_spec(32, 384), _full_spec(1, 384),
                  _full_spec(32, 384), _full_spec(1, 384)],
        out_specs=[_row_spec(384), _row_spec(384)],
        out_shape=[jax.ShapeDtypeStruct((N, 384), _f32),
                   jax.ShapeDtypeStruct((N, 384), _f32)],
    )(a0, a1, h2p, dinv, b2r, w0f, c0f, w0b, c0b)


def _gi1(f0, r0, wft, wfb, c1f, wbt, wbb, c1b):
    def body(fr, rr, wft_, wfb_, cf, wbt_, wbb_, cb, of, ob):
        f = fr[...]
        r = rr[...]
        of[...] = (jnp.dot(f, wft_[...], preferred_element_type=_f32)
                   + jnp.dot(r, wfb_[...], preferred_element_type=_f32)
                   + cf[...])
        ob[...] = (jnp.dot(f, wbt_[...], preferred_element_type=_f32)
                   + jnp.dot(r, wbb_[...], preferred_element_type=_f32)
                   + cb[...])

    return pl.pallas_call(
        body,
        grid=(_G,),
        in_specs=[_row_spec(64), _row_spec(64),
                  _full_spec(64, 384), _full_spec(64, 384), _full_spec(1, 384),
                  _full_spec(64, 384), _full_spec(64, 384), _full_spec(1, 384)],
        out_specs=[_row_spec(384), _row_spec(384)],
        out_shape=[jax.ShapeDtypeStruct((N, 384), _f32),
                   jax.ShapeDtypeStruct((N, 384), _f32)],
    )(f0, r0, wft, wfb, c1f, wbt, wbb, c1b)


def _gru(gif, gib, wblk, bn):
    def body(gif_ref, gib_ref, wblk_ref, bn_ref, f_ref, r_ref):
        w = wblk_ref[...]
        bnv = bn_ref[...]

        def step(t, h):
            g = gif_ref[pl.ds(t, 1), :] + gib_ref[pl.ds(N - 1 - t, 1), :]
            gh = jnp.dot(h.astype(jnp.bfloat16), w,
                         preferred_element_type=_f32)
            rg = 0.5 + 0.5 * jnp.tanh(0.5 * (g[:, 0:128] + gh[:, 0:128]))
            zg = 0.5 + 0.5 * jnp.tanh(0.5 * (g[:, 128:256] + gh[:, 128:256]))
            ng = jnp.tanh(g[:, 256:384] + rg * (gh[:, 256:384] + bnv))
            h2 = ng + zg * (h - ng)
            f_ref[pl.ds(t, 1), :] = h2[:, 0:64]
            r_ref[pl.ds(N - 1 - t, 1), :] = h2[:, 64:128]
            return h2

        lax.fori_loop(0, N, step, jnp.zeros((1, 128), _f32))

    return pl.pallas_call(
        body,
        out_shape=[jax.ShapeDtypeStruct((N, 64), _f32),
                   jax.ShapeDtypeStruct((N, 64), _f32)],
    )(gif, gib, wblk.astype(jnp.bfloat16), bn)


def _final(f1, r1, wlt, wlb, blr):
    def body(fr, rr, wt, wb, bref, out):
        out[...] = (jnp.dot(fr[...], wt[...], preferred_element_type=_f32)
                    + jnp.dot(rr[...], wb[...], preferred_element_type=_f32)
                    + bref[...])

    return pl.pallas_call(
        body,
        grid=(_G,),
        in_specs=[_row_spec(64), _row_spec(64),
                  _full_spec(64, 10), _full_spec(64, 10), _full_spec(1, 10)],
        out_specs=_row_spec(10),
        out_shape=jax.ShapeDtypeStruct((N, 10), _f32),
    )(f1, r1, wlt, wlb, blr)


def _pack_gi(Wih, bih, bhh, off):
    T = Wih.T.astype(_f32)
    inn = T.shape[0]
    w = jnp.zeros((inn, 384), _f32)
    w = w.at[:, off:off + 64].set(T[:, 0:64])
    w = w.at[:, 128 + off:128 + off + 64].set(T[:, 64:128])
    w = w.at[:, 256 + off:256 + off + 64].set(T[:, 128:192])
    cv = jnp.zeros((384,), _f32)
    cv = cv.at[off:off + 64].set(bih[0:64] + bhh[0:64])
    cv = cv.at[128 + off:128 + off + 64].set(bih[64:128] + bhh[64:128])
    cv = cv.at[256 + off:256 + off + 64].set(bih[128:192])
    return w, cv.reshape(1, 384)


def _pack_whh(Whh_f, Whh_b, bhh_f, bhh_b):
    w = jnp.zeros((128, 384), _f32)
    w = w.at[0:64, 0:64].set(Whh_f[0:64].T)
    w = w.at[64:128, 64:128].set(Whh_b[0:64].T)
    w = w.at[0:64, 128:192].set(Whh_f[64:128].T)
    w = w.at[64:128, 192:256].set(Whh_b[64:128].T)
    w = w.at[0:64, 256:320].set(Whh_f[128:192].T)
    w = w.at[64:128, 320:384].set(Whh_b[128:192].T)
    bn = jnp.concatenate([bhh_f[128:192], bhh_b[128:192]]).reshape(1, 128)
    return w, bn


def kernel(x, edge_index, W1, b1, W2, b2,
           l0f_Wih, l0f_Whh, l0f_bih, l0f_bhh,
           l0b_Wih, l0b_Whh, l0b_bih, l0b_bhh,
           l1f_Wih, l1f_Whh, l1f_bih, l1f_bhh,
           l1b_Wih, l1b_Whh, l1b_bih, l1b_bhh, Wl, bl):
    pad = EPAD - E
    src2 = jnp.concatenate(
        [edge_index[0], jnp.zeros((pad,), jnp.int32)]).reshape(EPAD // CW, CW)
    dst2 = jnp.concatenate(
        [edge_index[1], jnp.full((pad,), N, jnp.int32)]).reshape(EPAD // CW, CW)

    degp = _sc_degree(dst2)
    h1p, dinv = _prep1(degp[0, :N, 0:1], degp[1, :N, 0:1], x, W1)
    acc1 = _sc_scatter(h1p, src2, dst2, 16)
    h2p = _mid(acc1[0, :N], acc1[1, :N], h1p, dinv, b1.reshape(1, 16), W2)
    acc2 = _sc_scatter(h2p, src2, dst2, 32)

    w0f, c0f = _pack_gi(l0f_Wih, l0f_bih, l0f_bhh, 0)
    w0b, c0b = _pack_gi(l0b_Wih, l0b_bih, l0b_bhh, 64)
    gi0f, gi0b = _gi0(acc2[0, :N], acc2[1, :N], h2p, dinv,
                      b2.reshape(1, 32), w0f, c0f, w0b, c0b)
    wblk0, bn0 = _pack_whh(l0f_Whh, l0b_Whh, l0f_bhh, l0b_bhh)
    f0, r0 = _gru(gi0f, gi0b, wblk0, bn0)

    w1f, c1f = _pack_gi(l1f_Wih, l1f_bih, l1f_bhh, 0)
    w1b, c1b = _pack_gi(l1b_Wih, l1b_bih, l1b_bhh, 64)
    gi1f, gi1b = _gi1(f0, r0, w1f[0:64], w1f[64:128], c1f,
                      w1b[0:64], w1b[64:128], c1b)
    wblk1, bn1 = _pack_whh(l1f_Whh, l1b_Whh, l1f_bhh, l1b_bhh)
    f1, r1 = _gru(gi1f, gi1b, wblk1, bn1)

    return _final(f1, r1, Wl[0:64], Wl[64:128], bl.reshape(1, 10))


# speedup vs baseline: 14.3092x; 1.3791x over previous
import functools

import jax
import jax.numpy as jnp
from jax import lax
from jax.experimental import pallas as pl
from jax.experimental.pallas import tpu as pltpu
import jax.experimental.pallas.tpu_sc as plsc

N = 10000
E = 320000
NP = 10240
SROWS = NP // 16
CW = 80
NWORK = 32
KCH = 128
EPAD = NWORK * KCH * CW
GH = 64

_f32 = jnp.float32


def _mesh():
    return plsc.VectorSubcoreMesh(core_axis_name="c", subcore_axis_name="s")


def _sc_degree(dst2):
    @functools.partial(
        pl.kernel,
        out_type=jax.ShapeDtypeStruct((2, NP, 16), _f32),
        mesh=_mesh(),
        compiler_params=pltpu.CompilerParams(use_tc_tiling_on_sc=False),
        scratch_types=[
            pltpu.VMEM((KCH, CW), jnp.int32),
            pltpu.VMEM((CW, 16), _f32),
            pltpu.VMEM_SHARED((NP, 16), _f32),
        ],
    )
    def k(dst_hbm, ones_hbm, zeros_hbm, out_hbm, dst_v, ones_v, acc_sh):
        c = lax.axis_index("c")
        s = lax.axis_index("s")
        w = s * 2 + c
        pltpu.sync_copy(zeros_hbm.at[pl.ds(s * SROWS, SROWS)],
                        acc_sh.at[pl.ds(s * SROWS, SROWS)])
        pltpu.sync_copy(ones_hbm, ones_v)
        pltpu.sync_copy(dst_hbm.at[pl.ds(w * KCH, KCH)], dst_v)
        plsc.subcore_barrier()

        def body(j, carry):
            pltpu.sync_copy(ones_v, acc_sh.at[dst_v.at[j]], add=True)
            return carry

        lax.fori_loop(0, KCH, body, 0)
        plsc.subcore_barrier()
        pltpu.sync_copy(acc_sh.at[pl.ds(s * SROWS, SROWS)],
                        out_hbm.at[c, pl.ds(s * SROWS, SROWS)])

    return k(dst2, jnp.ones((CW, 16), _f32), jnp.zeros((NP, 16), _f32))


def _sc_scatter(hp, src2, dst2, width):
    @functools.partial(
        pl.kernel,
        out_type=jax.ShapeDtypeStruct((2, NP, width), _f32),
        mesh=_mesh(),
        compiler_params=pltpu.CompilerParams(use_tc_tiling_on_sc=False),
        scratch_types=[
            pltpu.VMEM((KCH, CW), jnp.int32),
            pltpu.VMEM((KCH, CW), jnp.int32),
            pltpu.VMEM((CW, width), _f32),
            pltpu.VMEM_SHARED((NP, width), _f32),
            pltpu.SemaphoreType.DMA,
        ],
    )
    def k(hp_hbm, src_hbm, dst_hbm, zeros_hbm, out_hbm,
          src_v, dst_v, rows_v, acc_sh, sem):
        c = lax.axis_index("c")
        s = lax.axis_index("s")
        w = s * 2 + c
        pltpu.sync_copy(zeros_hbm.at[pl.ds(s * SROWS, SROWS)],
                        acc_sh.at[pl.ds(s * SROWS, SROWS)])
        pltpu.sync_copy(src_hbm.at[pl.ds(w * KCH, KCH)], src_v)
        pltpu.sync_copy(dst_hbm.at[pl.ds(w * KCH, KCH)], dst_v)
        plsc.subcore_barrier()

        def body(j, carry):
            pltpu.async_copy(hp_hbm.at[src_v.at[j]], rows_v, sem).wait()
            pltpu.sync_copy(rows_v, acc_sh.at[dst_v.at[j]], add=True)
            return carry

        lax.fori_loop(0, KCH, body, 0)
        plsc.subcore_barrier()
        pltpu.sync_copy(acc_sh.at[pl.ds(s * SROWS, SROWS)],
                        out_hbm.at[c, pl.ds(s * SROWS, SROWS)])

    return k(hp, src2, dst2, jnp.zeros((NP, width), _f32))


_R = 1000
_G = N // _R


def _row_spec(width):
    return pl.BlockSpec((_R, width), lambda i: (i, 0))


def _full_spec(a, b):
    return pl.BlockSpec((a, b), lambda i: (0, 0))


def _prep1(dega, degb, x, W1):
    def body(da, db, xr, w1, h1p, dv):
        dinv = lax.rsqrt(da[...] + db[...] + 1.0)
        h1p[...] = jnp.dot(xr[...], w1[...],
                           preferred_element_type=_f32) * dinv
        dv[...] = dinv

    return pl.pallas_call(
        body,
        grid=(_G,),
        in_specs=[_row_spec(1), _row_spec(1), _row_spec(128),
                  _full_spec(128, 16)],
        out_specs=[_row_spec(16), _row_spec(1)],
        out_shape=[jax.ShapeDtypeStruct((N, 16), _f32),
                   jax.ShapeDtypeStruct((N, 1), _f32)],
    )(dega, degb, x, W1)


def _mid(a0, a1, h1p, dinv, b1r, W2):
    def body(a0r, a1r, hpr, dvr, b1ref, w2, out):
        h1 = jnp.maximum((a0r[...] + a1r[...] + hpr[...]) * dvr[...]
                         + b1ref[...], 0.0)
        out[...] = jnp.dot(h1, w2[...], preferred_element_type=_f32) * dvr[...]

    return pl.pallas_call(
        body,
        grid=(_G,),
        in_specs=[_row_spec(16), _row_spec(16), _row_spec(16), _row_spec(1),
                  _full_spec(1, 16), _full_spec(16, 32)],
        out_specs=_row_spec(32),
        out_shape=jax.ShapeDtypeStruct((N, 32), _f32),
    )(a0, a1, h1p, dinv, b1r, W2)


def _gi0(a0, a1, h2p, dinv, b2r, w0f, c0f, w0b, c0b):
    def body(a0r, a1r, hpr, dvr, b2ref, wf, cf, wb, cb, of, ob):
        h2 = jnp.maximum((a0r[...] + a1r[...] + hpr[...]) * dvr[...]
                         + b2ref[...], 0.0)
        of[...] = jnp.dot(h2, wf[...], preferred_element_type=_f32) + cf[...]
        ob[...] = jnp.dot(h2, wb[...], preferred_element_type=_f32) + cb[...]

    return pl.pallas_call(
        body,
        grid=(_G,),
        in_specs=[_row_spec(32), _row_spec(32), _row_spec(32), _row_spec(1),
                  _full_spec(1, 32), _full_spec(32, 384), _full_spec(1, 384),
                  _full_spec(32, 384), _full_spec(1, 384)],
        out_specs=[_row_spec(384), _row_spec(384)],
        out_shape=[jax.ShapeDtypeStruct((N, 384), _f32),
                   jax.ShapeDtypeStruct((N, 384), _f32)],
    )(a0, a1, h2p, dinv, b2r, w0f, c0f, w0b, c0b)


def _gi1(f0, r0, wft, wfb, c1f, wbt, wbb, c1b):
    def body(fr, rr, wft_, wfb_, cf, wbt_, wbb_, cb, of, ob):
        f = fr[...]
        r = rr[...]
        of[...] = (jnp.dot(f, wft_[...], preferred_element_type=_f32)
                   + jnp.dot(r, wfb_[...], preferred_element_type=_f32)
                   + cf[...])
        ob[...] = (jnp.dot(f, wbt_[...], preferred_element_type=_f32)
                   + jnp.dot(r, wbb_[...], preferred_element_type=_f32)
                   + cb[...])

    return pl.pallas_call(
        body,
        grid=(_G,),
        in_specs=[_row_spec(128), _row_spec(128),
                  _full_spec(128, 384), _full_spec(128, 384),
                  _full_spec(1, 384),
                  _full_spec(128, 384), _full_spec(128, 384),
                  _full_spec(1, 384)],
        out_specs=[_row_spec(384), _row_spec(384)],
        out_shape=[jax.ShapeDtypeStruct((N, 384), _f32),
                   jax.ShapeDtypeStruct((N, 384), _f32)],
    )(f0, r0, wft, wfb, c1f, wbt, wbb, c1b)


def _gru(gif, gib, wblk, bn):
    def body(gif_ref, gib_ref, wblk_ref, bn_ref, f_ref, r_ref):
        w = wblk_ref[...]
        bnv = bn_ref[...]

        def step(t, h):
            g = gif_ref[pl.ds(t, 1), :] + gib_ref[pl.ds(N - 1 - t, 1), :]
            gh = jnp.dot(h.astype(jnp.bfloat16), w,
                         preferred_element_type=_f32)
            rg = 0.5 + 0.5 * jnp.tanh(0.5 * (g[:, 0:128] + gh[:, 0:128]))
            zg = 0.5 + 0.5 * jnp.tanh(0.5 * (g[:, 128:256] + gh[:, 128:256]))
            ng = jnp.tanh(g[:, 256:384] + rg * (gh[:, 256:384] + bnv))
            h2 = ng + zg * (h - ng)
            f_ref[pl.ds(t, 1), :] = h2
            r_ref[pl.ds(N - 1 - t, 1), :] = h2
            return h2

        lax.fori_loop(0, N, step, jnp.zeros((1, 128), _f32))

    return pl.pallas_call(
        body,
        out_shape=[jax.ShapeDtypeStruct((N, 128), _f32),
                   jax.ShapeDtypeStruct((N, 128), _f32)],
    )(gif, gib, wblk.astype(jnp.bfloat16), bn)


def _final(f1, r1, wlt, wlb, blr):
    def body(fr, rr, wt, wb, bref, out):
        out[...] = (jnp.dot(fr[...], wt[...], preferred_element_type=_f32)
                    + jnp.dot(rr[...], wb[...], preferred_element_type=_f32)
                    + bref[...])

    return pl.pallas_call(
        body,
        grid=(_G,),
        in_specs=[_row_spec(128), _row_spec(128),
                  _full_spec(128, 10), _full_spec(128, 10), _full_spec(1, 10)],
        out_specs=_row_spec(10),
        out_shape=jax.ShapeDtypeStruct((N, 10), _f32),
    )(f1, r1, wlt, wlb, blr)


def _pack_gi(Wih, bih, bhh, off):
    T = Wih.T.astype(_f32)
    inn = T.shape[0]
    w = jnp.zeros((inn, 384), _f32)
    w = w.at[:, off:off + 64].set(T[:, 0:64])
    w = w.at[:, 128 + off:128 + off + 64].set(T[:, 64:128])
    w = w.at[:, 256 + off:256 + off + 64].set(T[:, 128:192])
    cv = jnp.zeros((384,), _f32)
    cv = cv.at[off:off + 64].set(bih[0:64] + bhh[0:64])
    cv = cv.at[128 + off:128 + off + 64].set(bih[64:128] + bhh[64:128])
    cv = cv.at[256 + off:256 + off + 64].set(bih[128:192])
    return w, cv.reshape(1, 384)


def _pack_whh(Whh_f, Whh_b, bhh_f, bhh_b):
    w = jnp.zeros((128, 384), _f32)
    w = w.at[0:64, 0:64].set(Whh_f[0:64].T)
    w = w.at[64:128, 64:128].set(Whh_b[0:64].T)
    w = w.at[0:64, 128:192].set(Whh_f[64:128].T)
    w = w.at[64:128, 192:256].set(Whh_b[64:128].T)
    w = w.at[0:64, 256:320].set(Whh_f[128:192].T)
    w = w.at[64:128, 320:384].set(Whh_b[128:192].T)
    bn = jnp.concatenate([bhh_f[128:192], bhh_b[128:192]]).reshape(1, 128)
    return w, bn


def kernel(x, edge_index, W1, b1, W2, b2,
           l0f_Wih, l0f_Whh, l0f_bih, l0f_bhh,
           l0b_Wih, l0b_Whh, l0b_bih, l0b_bhh,
           l1f_Wih, l1f_Whh, l1f_bih, l1f_bhh,
           l1b_Wih, l1b_Whh, l1b_bih, l1b_bhh, Wl, bl):
    pad = EPAD - E
    src2 = jnp.concatenate(
        [edge_index[0], jnp.zeros((pad,), jnp.int32)]).reshape(EPAD // CW, CW)
    dst2 = jnp.concatenate(
        [edge_index[1], jnp.full((pad,), N, jnp.int32)]).reshape(EPAD // CW, CW)

    degp = _sc_degree(dst2)
    h1p, dinv = _prep1(degp[0, :N, 0:1], degp[1, :N, 0:1], x, W1)
    acc1 = _sc_scatter(h1p, src2, dst2, 16)
    h2p = _mid(acc1[0, :N], acc1[1, :N], h1p, dinv, b1.reshape(1, 16), W2)
    acc2 = _sc_scatter(h2p, src2, dst2, 32)

    w0f, c0f = _pack_gi(l0f_Wih, l0f_bih, l0f_bhh, 0)
    w0b, c0b = _pack_gi(l0b_Wih, l0b_bih, l0b_bhh, 64)
    gi0f, gi0b = _gi0(acc2[0, :N], acc2[1, :N], h2p, dinv,
                      b2.reshape(1, 32), w0f, c0f, w0b, c0b)
    wblk0, bn0 = _pack_whh(l0f_Whh, l0b_Whh, l0f_bhh, l0b_bhh)
    f0, r0 = _gru(gi0f, gi0b, wblk0, bn0)

    z64 = jnp.zeros((64, 384), _f32)
    w1f, c1f = _pack_gi(l1f_Wih, l1f_bih, l1f_bhh, 0)
    w1b, c1b = _pack_gi(l1b_Wih, l1b_bih, l1b_bhh, 64)
    gi1f, gi1b = _gi1(f0, r0,
                      jnp.concatenate([w1f[0:64], z64]),
                      jnp.concatenate([z64, w1f[64:128]]), c1f,
                      jnp.concatenate([w1b[0:64], z64]),
                      jnp.concatenate([z64, w1b[64:128]]), c1b)
    wblk1, bn1 = _pack_whh(l1f_Whh, l1b_Whh, l1f_bhh, l1b_bhh)
    f1, r1 = _gru(gi1f, gi1b, wblk1, bn1)

    zl = jnp.zeros((64, 10), _f32)
    return _final(f1, r1,
                  jnp.concatenate([Wl[0:64], zl]),
                  jnp.concatenate([zl, Wl[64:128]]), bl.reshape(1, 10))


# speedup vs baseline: 14.6266x; 1.0222x over previous
import functools

import jax
import jax.numpy as jnp
from jax import lax
from jax.experimental import pallas as pl
from jax.experimental.pallas import tpu as pltpu
import jax.experimental.pallas.tpu_sc as plsc

N = 10000
E = 320000
NP = 10240
SROWS = NP // 16
CW = 128
NWORK = 32
KCH = 80
EPAD = NWORK * KCH * CW
GH = 64

_f32 = jnp.float32


def _mesh():
    return plsc.VectorSubcoreMesh(core_axis_name="c", subcore_axis_name="s")


def _sc_degree(dst2):
    @functools.partial(
        pl.kernel,
        out_type=jax.ShapeDtypeStruct((2, NP, 16), _f32),
        mesh=_mesh(),
        compiler_params=pltpu.CompilerParams(use_tc_tiling_on_sc=False),
        scratch_types=[
            pltpu.VMEM((KCH, CW), jnp.int32),
            pltpu.VMEM((CW, 16), _f32),
            pltpu.VMEM_SHARED((NP, 16), _f32),
        ],
    )
    def k(dst_hbm, ones_hbm, zeros_hbm, out_hbm, dst_v, ones_v, acc_sh):
        c = lax.axis_index("c")
        s = lax.axis_index("s")
        w = s * 2 + c
        pltpu.sync_copy(zeros_hbm.at[pl.ds(s * SROWS, SROWS)],
                        acc_sh.at[pl.ds(s * SROWS, SROWS)])
        pltpu.sync_copy(ones_hbm, ones_v)
        pltpu.sync_copy(dst_hbm.at[pl.ds(w * KCH, KCH)], dst_v)
        plsc.subcore_barrier()

        def body(j, carry):
            pltpu.sync_copy(ones_v, acc_sh.at[dst_v.at[j]], add=True)
            return carry

        lax.fori_loop(0, KCH, body, 0)
        plsc.subcore_barrier()
        pltpu.sync_copy(acc_sh.at[pl.ds(s * SROWS, SROWS)],
                        out_hbm.at[c, pl.ds(s * SROWS, SROWS)])

    return k(dst2, jnp.ones((CW, 16), _f32), jnp.zeros((NP, 16), _f32))


def _sc_scatter(hp, src2, dst2, width):
    @functools.partial(
        pl.kernel,
        out_type=jax.ShapeDtypeStruct((2, NP, width), _f32),
        mesh=_mesh(),
        compiler_params=pltpu.CompilerParams(use_tc_tiling_on_sc=False),
        scratch_types=[
            pltpu.VMEM((KCH, CW), jnp.int32),
            pltpu.VMEM((KCH, CW), jnp.int32),
            pltpu.VMEM((CW, width), _f32),
            pltpu.VMEM((CW, width), _f32),
            pltpu.VMEM_SHARED((NP, width), _f32),
            pltpu.SemaphoreType.DMA,
            pltpu.SemaphoreType.DMA,
        ],
    )
    def k(hp_hbm, src_hbm, dst_hbm, zeros_hbm, out_hbm,
          src_v, dst_v, rows_a, rows_b, acc_sh, sem_a, sem_b):
        c = lax.axis_index("c")
        s = lax.axis_index("s")
        w = s * 2 + c
        pltpu.sync_copy(zeros_hbm.at[pl.ds(s * SROWS, SROWS)],
                        acc_sh.at[pl.ds(s * SROWS, SROWS)])
        pltpu.sync_copy(src_hbm.at[pl.ds(w * KCH, KCH)], src_v)
        pltpu.sync_copy(dst_hbm.at[pl.ds(w * KCH, KCH)], dst_v)
        plsc.subcore_barrier()

        pltpu.async_copy(hp_hbm.at[src_v.at[0]], rows_a, sem_a)

        def body(j, carry):
            @pl.when(j % 2 == 0)
            def _():
                pltpu.make_async_copy(zeros_hbm.at[pl.ds(0, CW)], rows_a,
                                      sem_a).wait()
                @pl.when(j + 1 < KCH)
                def _():
                    pltpu.async_copy(hp_hbm.at[src_v.at[j + 1]], rows_b,
                                     sem_b)
                pltpu.sync_copy(rows_a, acc_sh.at[dst_v.at[j]], add=True)

            @pl.when(j % 2 == 1)
            def _():
                pltpu.make_async_copy(zeros_hbm.at[pl.ds(0, CW)], rows_b,
                                      sem_b).wait()
                @pl.when(j + 1 < KCH)
                def _():
                    pltpu.async_copy(hp_hbm.at[src_v.at[j + 1]], rows_a,
                                     sem_a)
                pltpu.sync_copy(rows_b, acc_sh.at[dst_v.at[j]], add=True)

            return carry

        lax.fori_loop(0, KCH, body, 0)
        plsc.subcore_barrier()
        pltpu.sync_copy(acc_sh.at[pl.ds(s * SROWS, SROWS)],
                        out_hbm.at[c, pl.ds(s * SROWS, SROWS)])

    return k(hp, src2, dst2, jnp.zeros((NP, width), _f32))


_R = 1000
_G = N // _R


def _row_spec(width):
    return pl.BlockSpec((_R, width), lambda i: (i, 0))


def _full_spec(a, b):
    return pl.BlockSpec((a, b), lambda i: (0, 0))


def _prep1(dega, degb, x, W1):
    def body(da, db, xr, w1, h1p, dv):
        dinv = lax.rsqrt(da[...] + db[...] + 1.0)
        h1p[...] = jnp.dot(xr[...], w1[...],
                           preferred_element_type=_f32) * dinv
        dv[...] = dinv

    return pl.pallas_call(
        body,
        grid=(_G,),
        in_specs=[_row_spec(1), _row_spec(1), _row_spec(128),
                  _full_spec(128, 16)],
        out_specs=[_row_spec(16), _row_spec(1)],
        out_shape=[jax.ShapeDtypeStruct((N, 16), _f32),
                   jax.ShapeDtypeStruct((N, 1), _f32)],
    )(dega, degb, x, W1)


def _mid(a0, a1, h1p, dinv, b1r, W2):
    def body(a0r, a1r, hpr, dvr, b1ref, w2, out):
        h1 = jnp.maximum((a0r[...] + a1r[...] + hpr[...]) * dvr[...]
                         + b1ref[...], 0.0)
        out[...] = jnp.dot(h1, w2[...], preferred_element_type=_f32) * dvr[...]

    return pl.pallas_call(
        body,
        grid=(_G,),
        in_specs=[_row_spec(16), _row_spec(16), _row_spec(16), _row_spec(1),
                  _full_spec(1, 16), _full_spec(16, 32)],
        out_specs=_row_spec(32),
        out_shape=jax.ShapeDtypeStruct((N, 32), _f32),
    )(a0, a1, h1p, dinv, b1r, W2)


def _gi0(a0, a1, h2p, dinv, b2r, w0f, c0f, w0b, c0b):
    def body(a0r, a1r, hpr, dvr, b2ref, wf, cf, wb, cb, of, ob):
        h2 = jnp.maximum((a0r[...] + a1r[...] + hpr[...]) * dvr[...]
                         + b2ref[...], 0.0)
        of[...] = jnp.dot(h2, wf[...], preferred_element_type=_f32) + cf[...]
        ob[...] = jnp.dot(h2, wb[...], preferred_element_type=_f32) + cb[...]

    return pl.pallas_call(
        body,
        grid=(_G,),
        in_specs=[_row_spec(32), _row_spec(32), _row_spec(32), _row_spec(1),
                  _full_spec(1, 32), _full_spec(32, 384), _full_spec(1, 384),
                  _full_spec(32, 384), _full_spec(1, 384)],
        out_specs=[_row_spec(384), _row_spec(384)],
        out_shape=[jax.ShapeDtypeStruct((N, 384), _f32),
                   jax.ShapeDtypeStruct((N, 384), _f32)],
    )(a0, a1, h2p, dinv, b2r, w0f, c0f, w0b, c0b)


def _gi1(f0, r0, wft, wfb, c1f, wbt, wbb, c1b):
    def body(fr, rr, wft_, wfb_, cf, wbt_, wbb_, cb, of, ob):
        f = fr[...]
        r = rr[...]
        of[...] = (jnp.dot(f, wft_[...], preferred_element_type=_f32)
                   + jnp.dot(r, wfb_[...], preferred_element_type=_f32)
                   + cf[...])
        ob[...] = (jnp.dot(f, wbt_[...], preferred_element_type=_f32)
                   + jnp.dot(r, wbb_[...], preferred_element_type=_f32)
                   + cb[...])

    return pl.pallas_call(
        body,
        grid=(_G,),
        in_specs=[_row_spec(128), _row_spec(128),
                  _full_spec(128, 384), _full_spec(128, 384),
                  _full_spec(1, 384),
                  _full_spec(128, 384), _full_spec(128, 384),
                  _full_spec(1, 384)],
        out_specs=[_row_spec(384), _row_spec(384)],
        out_shape=[jax.ShapeDtypeStruct((N, 384), _f32),
                   jax.ShapeDtypeStruct((N, 384), _f32)],
    )(f0, r0, wft, wfb, c1f, wbt, wbb, c1b)


def _gru(gif, gib, wblk, bn):
    def body(gif_ref, gib_ref, wblk_ref, bn_ref, f_ref, r_ref):
        w = wblk_ref[...]
        bnv = bn_ref[...]

        def step(t, h):
            g = gif_ref[pl.ds(t, 1), :] + gib_ref[pl.ds(N - 1 - t, 1), :]
            gh = jnp.dot(h.astype(jnp.bfloat16), w,
                         preferred_element_type=_f32)
            rg = 0.5 + 0.5 * jnp.tanh(0.5 * (g[:, 0:128] + gh[:, 0:128]))
            zg = 0.5 + 0.5 * jnp.tanh(0.5 * (g[:, 128:256] + gh[:, 128:256]))
            ng = jnp.tanh(g[:, 256:384] + rg * (gh[:, 256:384] + bnv))
            h2 = ng + zg * (h - ng)
            f_ref[pl.ds(t, 1), :] = h2
            r_ref[pl.ds(N - 1 - t, 1), :] = h2
            return h2

        lax.fori_loop(0, N, step, jnp.zeros((1, 128), _f32))

    return pl.pallas_call(
        body,
        out_shape=[jax.ShapeDtypeStruct((N, 128), _f32),
                   jax.ShapeDtypeStruct((N, 128), _f32)],
    )(gif, gib, wblk.astype(jnp.bfloat16), bn)


def _final(f1, r1, wlt, wlb, blr):
    def body(fr, rr, wt, wb, bref, out):
        out[...] = (jnp.dot(fr[...], wt[...], preferred_element_type=_f32)
                    + jnp.dot(rr[...], wb[...], preferred_element_type=_f32)
                    + bref[...])

    return pl.pallas_call(
        body,
        grid=(_G,),
        in_specs=[_row_spec(128), _row_spec(128),
                  _full_spec(128, 10), _full_spec(128, 10), _full_spec(1, 10)],
        out_specs=_row_spec(10),
        out_shape=jax.ShapeDtypeStruct((N, 10), _f32),
    )(f1, r1, wlt, wlb, blr)


def _pack_gi(Wih, bih, bhh, off):
    T = Wih.T.astype(_f32)
    inn = T.shape[0]
    w = jnp.zeros((inn, 384), _f32)
    w = w.at[:, off:off + 64].set(T[:, 0:64])
    w = w.at[:, 128 + off:128 + off + 64].set(T[:, 64:128])
    w = w.at[:, 256 + off:256 + off + 64].set(T[:, 128:192])
    cv = jnp.zeros((384,), _f32)
    cv = cv.at[off:off + 64].set(bih[0:64] + bhh[0:64])
    cv = cv.at[128 + off:128 + off + 64].set(bih[64:128] + bhh[64:128])
    cv = cv.at[256 + off:256 + off + 64].set(bih[128:192])
    return w, cv.reshape(1, 384)


def _pack_whh(Whh_f, Whh_b, bhh_f, bhh_b):
    w = jnp.zeros((128, 384), _f32)
    w = w.at[0:64, 0:64].set(Whh_f[0:64].T)
    w = w.at[64:128, 64:128].set(Whh_b[0:64].T)
    w = w.at[0:64, 128:192].set(Whh_f[64:128].T)
    w = w.at[64:128, 192:256].set(Whh_b[64:128].T)
    w = w.at[0:64, 256:320].set(Whh_f[128:192].T)
    w = w.at[64:128, 320:384].set(Whh_b[128:192].T)
    bn = jnp.concatenate([bhh_f[128:192], bhh_b[128:192]]).reshape(1, 128)
    return w, bn


def kernel(x, edge_index, W1, b1, W2, b2,
           l0f_Wih, l0f_Whh, l0f_bih, l0f_bhh,
           l0b_Wih, l0b_Whh, l0b_bih, l0b_bhh,
           l1f_Wih, l1f_Whh, l1f_bih, l1f_bhh,
           l1b_Wih, l1b_Whh, l1b_bih, l1b_bhh, Wl, bl):
    pad = EPAD - E
    src2 = jnp.concatenate(
        [edge_index[0], jnp.zeros((pad,), jnp.int32)]).reshape(EPAD // CW, CW)
    dst2 = jnp.concatenate(
        [edge_index[1], jnp.full((pad,), N, jnp.int32)]).reshape(EPAD // CW, CW)

    degp = _sc_degree(dst2)
    h1p, dinv = _prep1(degp[0, :N, 0:1], degp[1, :N, 0:1], x, W1)
    acc1 = _sc_scatter(h1p, src2, dst2, 16)
    h2p = _mid(acc1[0, :N], acc1[1, :N], h1p, dinv, b1.reshape(1, 16), W2)
    acc2 = _sc_scatter(h2p, src2, dst2, 32)

    w0f, c0f = _pack_gi(l0f_Wih, l0f_bih, l0f_bhh, 0)
    w0b, c0b = _pack_gi(l0b_Wih, l0b_bih, l0b_bhh, 64)
    gi0f, gi0b = _gi0(acc2[0, :N], acc2[1, :N], h2p, dinv,
                      b2.reshape(1, 32), w0f, c0f, w0b, c0b)
    wblk0, bn0 = _pack_whh(l0f_Whh, l0b_Whh, l0f_bhh, l0b_bhh)
    f0, r0 = _gru(gi0f, gi0b, wblk0, bn0)

    z64 = jnp.zeros((64, 384), _f32)
    w1f, c1f = _pack_gi(l1f_Wih, l1f_bih, l1f_bhh, 0)
    w1b, c1b = _pack_gi(l1b_Wih, l1b_bih, l1b_bhh, 64)
    gi1f, gi1b = _gi1(f0, r0,
                      jnp.concatenate([w1f[0:64], z64]),
                      jnp.concatenate([z64, w1f[64:128]]), c1f,
                      jnp.concatenate([w1b[0:64], z64]),
                      jnp.concatenate([z64, w1b[64:128]]), c1b)
    wblk1, bn1 = _pack_whh(l1f_Whh, l1b_Whh, l1f_bhh, l1b_bhh)
    f1, r1 = _gru(gi1f, gi1b, wblk1, bn1)

    zl = jnp.zeros((64, 10), _f32)
    return _final(f1, r1,
                  jnp.concatenate([Wl[0:64], zl]),
                  jnp.concatenate([zl, Wl[64:128]]), bl.reshape(1, 10))


# speedup vs baseline: 15.3565x; 1.0499x over previous
import functools

import jax
import jax.numpy as jnp
from jax import lax
from jax.experimental import pallas as pl
from jax.experimental.pallas import tpu as pltpu
import jax.experimental.pallas.tpu_sc as plsc

N = 10000
E = 320000
NP = 10240
SROWS = NP // 16
CW = 128
NWORK = 32
KCH = 80
EPAD = NWORK * KCH * CW
GH = 64

_f32 = jnp.float32


def _mesh():
    return plsc.VectorSubcoreMesh(core_axis_name="c", subcore_axis_name="s")


def _sc_degree(dst2):
    @functools.partial(
        pl.kernel,
        out_type=jax.ShapeDtypeStruct((2, NP, 16), _f32),
        mesh=_mesh(),
        compiler_params=pltpu.CompilerParams(use_tc_tiling_on_sc=False),
        scratch_types=[
            pltpu.VMEM((KCH, CW), jnp.int32),
            pltpu.VMEM((CW, 16), _f32),
            pltpu.VMEM_SHARED((NP, 16), _f32),
        ],
    )
    def k(dst_hbm, ones_hbm, zeros_hbm, out_hbm, dst_v, ones_v, acc_sh):
        c = lax.axis_index("c")
        s = lax.axis_index("s")
        w = s * 2 + c
        pltpu.sync_copy(zeros_hbm.at[pl.ds(s * SROWS, SROWS)],
                        acc_sh.at[pl.ds(s * SROWS, SROWS)])
        pltpu.sync_copy(ones_hbm, ones_v)
        pltpu.sync_copy(dst_hbm.at[pl.ds(w * KCH, KCH)], dst_v)
        plsc.subcore_barrier()

        def body(j, carry):
            pltpu.sync_copy(ones_v, acc_sh.at[dst_v.at[j]], add=True)
            return carry

        lax.fori_loop(0, KCH, body, 0)
        plsc.subcore_barrier()
        pltpu.sync_copy(acc_sh.at[pl.ds(s * SROWS, SROWS)],
                        out_hbm.at[c, pl.ds(s * SROWS, SROWS)])

    return k(dst2, jnp.ones((CW, 16), _f32), jnp.zeros((NP, 16), _f32))


def _sc_scatter(hp, src2, dst2, width):
    @functools.partial(
        pl.kernel,
        out_type=jax.ShapeDtypeStruct((2, NP, width), _f32),
        mesh=_mesh(),
        compiler_params=pltpu.CompilerParams(use_tc_tiling_on_sc=False),
        scratch_types=[
            pltpu.VMEM((KCH, CW), jnp.int32),
            pltpu.VMEM((KCH, CW), jnp.int32),
            pltpu.VMEM((CW, width), _f32),
            pltpu.VMEM((CW, width), _f32),
            pltpu.VMEM_SHARED((NP, width), _f32),
            pltpu.SemaphoreType.DMA,
            pltpu.SemaphoreType.DMA,
        ],
    )
    def k(hp_hbm, src_hbm, dst_hbm, zeros_hbm, out_hbm,
          src_v, dst_v, rows_a, rows_b, acc_sh, sem_a, sem_b):
        c = lax.axis_index("c")
        s = lax.axis_index("s")
        w = s * 2 + c
        pltpu.sync_copy(zeros_hbm.at[pl.ds(s * SROWS, SROWS)],
                        acc_sh.at[pl.ds(s * SROWS, SROWS)])
        pltpu.sync_copy(src_hbm.at[pl.ds(w * KCH, KCH)], src_v)
        pltpu.sync_copy(dst_hbm.at[pl.ds(w * KCH, KCH)], dst_v)
        plsc.subcore_barrier()

        pltpu.async_copy(hp_hbm.at[src_v.at[0]], rows_a, sem_a)

        def body(j, carry):
            @pl.when(j % 2 == 0)
            def _():
                pltpu.make_async_copy(zeros_hbm.at[pl.ds(0, CW)], rows_a,
                                      sem_a).wait()
                @pl.when(j + 1 < KCH)
                def _():
                    pltpu.async_copy(hp_hbm.at[src_v.at[j + 1]], rows_b,
                                     sem_b)
                pltpu.sync_copy(rows_a, acc_sh.at[dst_v.at[j]], add=True)

            @pl.when(j % 2 == 1)
            def _():
                pltpu.make_async_copy(zeros_hbm.at[pl.ds(0, CW)], rows_b,
                                      sem_b).wait()
                @pl.when(j + 1 < KCH)
                def _():
                    pltpu.async_copy(hp_hbm.at[src_v.at[j + 1]], rows_a,
                                     sem_a)
                pltpu.sync_copy(rows_b, acc_sh.at[dst_v.at[j]], add=True)

            return carry

        lax.fori_loop(0, KCH, body, 0)
        plsc.subcore_barrier()
        pltpu.sync_copy(acc_sh.at[pl.ds(s * SROWS, SROWS)],
                        out_hbm.at[c, pl.ds(s * SROWS, SROWS)])

    return k(hp, src2, dst2, jnp.zeros((NP, width), _f32))


_R = 1000
_G = N // _R


def _row_spec(width):
    return pl.BlockSpec((_R, width), lambda i: (i, 0))


def _full_spec(a, b):
    return pl.BlockSpec((a, b), lambda i: (0, 0))


def _prep1(dega, degb, x, W1):
    def body(da, db, xr, w1, h1p, dv):
        dinv = lax.rsqrt(da[...] + db[...] + 1.0)
        h1p[...] = jnp.dot(xr[...], w1[...],
                           preferred_element_type=_f32) * dinv
        dv[...] = dinv

    return pl.pallas_call(
        body,
        grid=(_G,),
        in_specs=[_row_spec(1), _row_spec(1), _row_spec(128),
                  _full_spec(128, 16)],
        out_specs=[_row_spec(16), _row_spec(1)],
        out_shape=[jax.ShapeDtypeStruct((N, 16), _f32),
                   jax.ShapeDtypeStruct((N, 1), _f32)],
    )(dega, degb, x, W1)


def _mid(a0, a1, h1p, dinv, b1r, W2):
    def body(a0r, a1r, hpr, dvr, b1ref, w2, out):
        h1 = jnp.maximum((a0r[...] + a1r[...] + hpr[...]) * dvr[...]
                         + b1ref[...], 0.0)
        out[...] = jnp.dot(h1, w2[...], preferred_element_type=_f32) * dvr[...]

    return pl.pallas_call(
        body,
        grid=(_G,),
        in_specs=[_row_spec(16), _row_spec(16), _row_spec(16), _row_spec(1),
                  _full_spec(1, 16), _full_spec(16, 32)],
        out_specs=_row_spec(32),
        out_shape=jax.ShapeDtypeStruct((N, 32), _f32),
    )(a0, a1, h1p, dinv, b1r, W2)


def _gi0(a0, a1, h2p, dinv, b2r, w0f, c0f, w0b, c0b):
    def body(a0r, a1r, hpr, dvr, b2ref, wf, cf, wb, cb, of, ob):
        h2 = jnp.maximum((a0r[...] + a1r[...] + hpr[...]) * dvr[...]
                         + b2ref[...], 0.0)
        of[...] = jnp.dot(h2, wf[...], preferred_element_type=_f32) + cf[...]
        ob[...] = jnp.dot(h2, wb[...], preferred_element_type=_f32) + cb[...]

    return pl.pallas_call(
        body,
        grid=(_G,),
        in_specs=[_row_spec(32), _row_spec(32), _row_spec(32), _row_spec(1),
                  _full_spec(1, 32), _full_spec(32, 384), _full_spec(1, 384),
                  _full_spec(32, 384), _full_spec(1, 384)],
        out_specs=[_row_spec(384), _row_spec(384)],
        out_shape=[jax.ShapeDtypeStruct((N, 384), _f32),
                   jax.ShapeDtypeStruct((N, 384), _f32)],
    )(a0, a1, h2p, dinv, b2r, w0f, c0f, w0b, c0b)


def _gi1(f0, r0, wft, wfb, c1f, wbt, wbb, c1b):
    def body(fr, rr, wft_, wfb_, cf, wbt_, wbb_, cb, of, ob):
        f = fr[...]
        r = rr[...]
        of[...] = (jnp.dot(f, wft_[...], preferred_element_type=_f32)
                   + jnp.dot(r, wfb_[...], preferred_element_type=_f32)
                   + cf[...])
        ob[...] = (jnp.dot(f, wbt_[...], preferred_element_type=_f32)
                   + jnp.dot(r, wbb_[...], preferred_element_type=_f32)
                   + cb[...])

    return pl.pallas_call(
        body,
        grid=(_G,),
        in_specs=[_row_spec(128), _row_spec(128),
                  _full_spec(128, 384), _full_spec(128, 384),
                  _full_spec(1, 384),
                  _full_spec(128, 384), _full_spec(128, 384),
                  _full_spec(1, 384)],
        out_specs=[_row_spec(384), _row_spec(384)],
        out_shape=[jax.ShapeDtypeStruct((N, 384), _f32),
                   jax.ShapeDtypeStruct((N, 384), _f32)],
    )(f0, r0, wft, wfb, c1f, wbt, wbb, c1b)


def _gru(gif, gib, wblk, bn):
    def body(gif_ref, gib_ref, wblk_ref, bn_ref, f_ref, r_ref):
        w = wblk_ref[...]
        bnv = bn_ref[...]

        def step(t, h):
            g = gif_ref[pl.ds(t, 1), :] + gib_ref[pl.ds(N - 1 - t, 1), :]
            gh = jnp.dot(h.astype(jnp.bfloat16), w,
                         preferred_element_type=_f32)
            rg = 0.5 + 0.5 * jnp.tanh(0.5 * (g[:, 0:128] + gh[:, 0:128]))
            zg = 0.5 + 0.5 * jnp.tanh(0.5 * (g[:, 128:256] + gh[:, 128:256]))
            ng = jnp.tanh(g[:, 256:384] + rg * (gh[:, 256:384] + bnv))
            h2 = ng + zg * (h - ng)
            f_ref[pl.ds(t, 1), :] = h2
            r_ref[pl.ds(N - 1 - t, 1), :] = h2
            return h2

        def step2(i, h):
            return step(2 * i + 1, step(2 * i, h))

        lax.fori_loop(0, N // 2, step2, jnp.zeros((1, 128), _f32))

    return pl.pallas_call(
        body,
        out_shape=[jax.ShapeDtypeStruct((N, 128), _f32),
                   jax.ShapeDtypeStruct((N, 128), _f32)],
    )(gif, gib, wblk.astype(jnp.bfloat16), bn)


def _final(f1, r1, wlt, wlb, blr):
    def body(fr, rr, wt, wb, bref, out):
        out[...] = (jnp.dot(fr[...], wt[...], preferred_element_type=_f32)
                    + jnp.dot(rr[...], wb[...], preferred_element_type=_f32)
                    + bref[...])

    return pl.pallas_call(
        body,
        grid=(_G,),
        in_specs=[_row_spec(128), _row_spec(128),
                  _full_spec(128, 10), _full_spec(128, 10), _full_spec(1, 10)],
        out_specs=_row_spec(10),
        out_shape=jax.ShapeDtypeStruct((N, 10), _f32),
    )(f1, r1, wlt, wlb, blr)


def _pack_gi(Wih, bih, bhh, off):
    T = Wih.T.astype(_f32)
    inn = T.shape[0]
    w = jnp.zeros((inn, 384), _f32)
    w = w.at[:, off:off + 64].set(T[:, 0:64])
    w = w.at[:, 128 + off:128 + off + 64].set(T[:, 64:128])
    w = w.at[:, 256 + off:256 + off + 64].set(T[:, 128:192])
    cv = jnp.zeros((384,), _f32)
    cv = cv.at[off:off + 64].set(bih[0:64] + bhh[0:64])
    cv = cv.at[128 + off:128 + off + 64].set(bih[64:128] + bhh[64:128])
    cv = cv.at[256 + off:256 + off + 64].set(bih[128:192])
    return w, cv.reshape(1, 384)


def _pack_whh(Whh_f, Whh_b, bhh_f, bhh_b):
    w = jnp.zeros((128, 384), _f32)
    w = w.at[0:64, 0:64].set(Whh_f[0:64].T)
    w = w.at[64:128, 64:128].set(Whh_b[0:64].T)
    w = w.at[0:64, 128:192].set(Whh_f[64:128].T)
    w = w.at[64:128, 192:256].set(Whh_b[64:128].T)
    w = w.at[0:64, 256:320].set(Whh_f[128:192].T)
    w = w.at[64:128, 320:384].set(Whh_b[128:192].T)
    bn = jnp.concatenate([bhh_f[128:192], bhh_b[128:192]]).reshape(1, 128)
    return w, bn


def kernel(x, edge_index, W1, b1, W2, b2,
           l0f_Wih, l0f_Whh, l0f_bih, l0f_bhh,
           l0b_Wih, l0b_Whh, l0b_bih, l0b_bhh,
           l1f_Wih, l1f_Whh, l1f_bih, l1f_bhh,
           l1b_Wih, l1b_Whh, l1b_bih, l1b_bhh, Wl, bl):
    pad = EPAD - E
    src2 = jnp.concatenate(
        [edge_index[0], jnp.zeros((pad,), jnp.int32)]).reshape(EPAD // CW, CW)
    dst2 = jnp.concatenate(
        [edge_index[1], jnp.full((pad,), N, jnp.int32)]).reshape(EPAD // CW, CW)

    degp = _sc_degree(dst2)
    h1p, dinv = _prep1(degp[0, :N, 0:1], degp[1, :N, 0:1], x, W1)
    acc1 = _sc_scatter(h1p, src2, dst2, 16)
    h2p = _mid(acc1[0, :N], acc1[1, :N], h1p, dinv, b1.reshape(1, 16), W2)
    acc2 = _sc_scatter(h2p, src2, dst2, 32)

    w0f, c0f = _pack_gi(l0f_Wih, l0f_bih, l0f_bhh, 0)
    w0b, c0b = _pack_gi(l0b_Wih, l0b_bih, l0b_bhh, 64)
    gi0f, gi0b = _gi0(acc2[0, :N], acc2[1, :N], h2p, dinv,
                      b2.reshape(1, 32), w0f, c0f, w0b, c0b)
    wblk0, bn0 = _pack_whh(l0f_Whh, l0b_Whh, l0f_bhh, l0b_bhh)
    f0, r0 = _gru(gi0f, gi0b, wblk0, bn0)

    z64 = jnp.zeros((64, 384), _f32)
    w1f, c1f = _pack_gi(l1f_Wih, l1f_bih, l1f_bhh, 0)
    w1b, c1b = _pack_gi(l1b_Wih, l1b_bih, l1b_bhh, 64)
    gi1f, gi1b = _gi1(f0, r0,
                      jnp.concatenate([w1f[0:64], z64]),
                      jnp.concatenate([z64, w1f[64:128]]), c1f,
                      jnp.concatenate([w1b[0:64], z64]),
                      jnp.concatenate([z64, w1b[64:128]]), c1b)
    wblk1, bn1 = _pack_whh(l1f_Whh, l1b_Whh, l1f_bhh, l1b_bhh)
    f1, r1 = _gru(gi1f, gi1b, wblk1, bn1)

    zl = jnp.zeros((64, 10), _f32)
    return _final(f1, r1,
                  jnp.concatenate([Wl[0:64], zl]),
                  jnp.concatenate([zl, Wl[64:128]]), bl.reshape(1, 10))


# speedup vs baseline: 15.7272x; 1.0241x over previous
import functools

import jax
import jax.numpy as jnp
from jax import lax
from jax.experimental import pallas as pl
from jax.experimental.pallas import tpu as pltpu
import jax.experimental.pallas.tpu_sc as plsc

N = 10000
E = 320000
NP = 10240
SROWS = NP // 16
CW = 128
NWORK = 32
KCH = 80
EPAD = NWORK * KCH * CW
GH = 64

_f32 = jnp.float32


def _mesh():
    return plsc.VectorSubcoreMesh(core_axis_name="c", subcore_axis_name="s")


def _sc_degree(dst2):
    @functools.partial(
        pl.kernel,
        out_type=jax.ShapeDtypeStruct((2, NP, 16), _f32),
        mesh=_mesh(),
        compiler_params=pltpu.CompilerParams(use_tc_tiling_on_sc=False),
        scratch_types=[
            pltpu.VMEM((KCH, CW), jnp.int32),
            pltpu.VMEM((CW, 16), _f32),
            pltpu.VMEM_SHARED((NP, 16), _f32),
        ],
    )
    def k(dst_hbm, ones_hbm, zeros_hbm, out_hbm, dst_v, ones_v, acc_sh):
        c = lax.axis_index("c")
        s = lax.axis_index("s")
        w = s * 2 + c
        pltpu.sync_copy(zeros_hbm.at[pl.ds(s * SROWS, SROWS)],
                        acc_sh.at[pl.ds(s * SROWS, SROWS)])
        pltpu.sync_copy(ones_hbm, ones_v)
        pltpu.sync_copy(dst_hbm.at[pl.ds(w * KCH, KCH)], dst_v)
        plsc.subcore_barrier()

        def body(j, carry):
            pltpu.sync_copy(ones_v, acc_sh.at[dst_v.at[j]], add=True)
            return carry

        lax.fori_loop(0, KCH, body, 0)
        plsc.subcore_barrier()
        pltpu.sync_copy(acc_sh.at[pl.ds(s * SROWS, SROWS)],
                        out_hbm.at[c, pl.ds(s * SROWS, SROWS)])

    return k(dst2, jnp.ones((CW, 16), _f32), jnp.zeros((NP, 16), _f32))


def _sc_scatter(hp, src2, dst2, width):
    @functools.partial(
        pl.kernel,
        out_type=jax.ShapeDtypeStruct((2, NP, width), _f32),
        mesh=_mesh(),
        compiler_params=pltpu.CompilerParams(use_tc_tiling_on_sc=False),
        scratch_types=[
            pltpu.VMEM((KCH, CW), jnp.int32),
            pltpu.VMEM((KCH, CW), jnp.int32),
            pltpu.VMEM((CW, width), _f32),
            pltpu.VMEM((CW, width), _f32),
            pltpu.VMEM_SHARED((NP, width), _f32),
            pltpu.SemaphoreType.DMA,
            pltpu.SemaphoreType.DMA,
        ],
    )
    def k(hp_hbm, src_hbm, dst_hbm, zeros_hbm, out_hbm,
          src_v, dst_v, rows_a, rows_b, acc_sh, sem_a, sem_b):
        c = lax.axis_index("c")
        s = lax.axis_index("s")
        w = s * 2 + c
        pltpu.sync_copy(zeros_hbm.at[pl.ds(s * SROWS, SROWS)],
                        acc_sh.at[pl.ds(s * SROWS, SROWS)])
        pltpu.sync_copy(src_hbm.at[pl.ds(w * KCH, KCH)], src_v)
        pltpu.sync_copy(dst_hbm.at[pl.ds(w * KCH, KCH)], dst_v)
        plsc.subcore_barrier()

        pltpu.async_copy(hp_hbm.at[src_v.at[0]], rows_a, sem_a)

        def body(j, carry):
            @pl.when(j % 2 == 0)
            def _():
                pltpu.make_async_copy(zeros_hbm.at[pl.ds(0, CW)], rows_a,
                                      sem_a).wait()
                @pl.when(j + 1 < KCH)
                def _():
                    pltpu.async_copy(hp_hbm.at[src_v.at[j + 1]], rows_b,
                                     sem_b)
                pltpu.sync_copy(rows_a, acc_sh.at[dst_v.at[j]], add=True)

            @pl.when(j % 2 == 1)
            def _():
                pltpu.make_async_copy(zeros_hbm.at[pl.ds(0, CW)], rows_b,
                                      sem_b).wait()
                @pl.when(j + 1 < KCH)
                def _():
                    pltpu.async_copy(hp_hbm.at[src_v.at[j + 1]], rows_a,
                                     sem_a)
                pltpu.sync_copy(rows_b, acc_sh.at[dst_v.at[j]], add=True)

            return carry

        lax.fori_loop(0, KCH, body, 0)
        plsc.subcore_barrier()
        pltpu.sync_copy(acc_sh.at[pl.ds(s * SROWS, SROWS)],
                        out_hbm.at[c, pl.ds(s * SROWS, SROWS)])

    return k(hp, src2, dst2, jnp.zeros((NP, width), _f32))


_R = 1000
_G = N // _R


def _row_spec(width):
    return pl.BlockSpec((_R, width), lambda i: (i, 0))


def _full_spec(a, b):
    return pl.BlockSpec((a, b), lambda i: (0, 0))


def _prep1(dega, degb, x, W1):
    def body(da, db, xr, w1, h1p, dv):
        dinv = lax.rsqrt(da[...] + db[...] + 1.0)
        h1p[...] = jnp.dot(xr[...], w1[...],
                           preferred_element_type=_f32) * dinv
        dv[...] = dinv

    return pl.pallas_call(
        body,
        grid=(_G,),
        in_specs=[_row_spec(1), _row_spec(1), _row_spec(128),
                  _full_spec(128, 16)],
        out_specs=[_row_spec(16), _row_spec(1)],
        out_shape=[jax.ShapeDtypeStruct((N, 16), _f32),
                   jax.ShapeDtypeStruct((N, 1), _f32)],
    )(dega, degb, x, W1)


def _mid(a0, a1, h1p, dinv, b1r, W2):
    def body(a0r, a1r, hpr, dvr, b1ref, w2, out):
        h1 = jnp.maximum((a0r[...] + a1r[...] + hpr[...]) * dvr[...]
                         + b1ref[...], 0.0)
        out[...] = jnp.dot(h1, w2[...], preferred_element_type=_f32) * dvr[...]

    return pl.pallas_call(
        body,
        grid=(_G,),
        in_specs=[_row_spec(16), _row_spec(16), _row_spec(16), _row_spec(1),
                  _full_spec(1, 16), _full_spec(16, 32)],
        out_specs=_row_spec(32),
        out_shape=jax.ShapeDtypeStruct((N, 32), _f32),
    )(a0, a1, h1p, dinv, b1r, W2)


def _gi0(a0, a1, h2p, dinv, b2r, w0f, c0f, w0b, c0b):
    def body(a0r, a1r, hpr, dvr, b2ref, wf, cf, wb, cb, of, ob):
        h2 = jnp.maximum((a0r[...] + a1r[...] + hpr[...]) * dvr[...]
                         + b2ref[...], 0.0)
        of[...] = jnp.dot(h2, wf[...], preferred_element_type=_f32) + cf[...]
        ob[...] = jnp.dot(h2, wb[...], preferred_element_type=_f32) + cb[...]

    return pl.pallas_call(
        body,
        grid=(_G,),
        in_specs=[_row_spec(32), _row_spec(32), _row_spec(32), _row_spec(1),
                  _full_spec(1, 32), _full_spec(32, 384), _full_spec(1, 384),
                  _full_spec(32, 384), _full_spec(1, 384)],
        out_specs=[_row_spec(384), _row_spec(384)],
        out_shape=[jax.ShapeDtypeStruct((N, 384), _f32),
                   jax.ShapeDtypeStruct((N, 384), _f32)],
    )(a0, a1, h2p, dinv, b2r, w0f, c0f, w0b, c0b)


def _gi1(f0, r0, wft, wfb, c1f, wbt, wbb, c1b):
    def body(fr, rr, wft_, wfb_, cf, wbt_, wbb_, cb, of, ob):
        f = fr[...]
        r = rr[...]
        of[...] = (jnp.dot(f, wft_[...], preferred_element_type=_f32)
                   + jnp.dot(r, wfb_[...], preferred_element_type=_f32)
                   + cf[...])
        ob[...] = (jnp.dot(f, wbt_[...], preferred_element_type=_f32)
                   + jnp.dot(r, wbb_[...], preferred_element_type=_f32)
                   + cb[...])

    return pl.pallas_call(
        body,
        grid=(_G,),
        in_specs=[_row_spec(128), _row_spec(128),
                  _full_spec(128, 384), _full_spec(128, 384),
                  _full_spec(1, 384),
                  _full_spec(128, 384), _full_spec(128, 384),
                  _full_spec(1, 384)],
        out_specs=[_row_spec(384), _row_spec(384)],
        out_shape=[jax.ShapeDtypeStruct((N, 384), _f32),
                   jax.ShapeDtypeStruct((N, 384), _f32)],
    )(f0, r0, wft, wfb, c1f, wbt, wbb, c1b)


def _gru(gif, gib, wblk, bn):
    def body(gif_ref, gib_ref, wblk_ref, bn_ref, f_ref, r_ref):
        w = wblk_ref[...]
        bnv = bn_ref[...]

        def step(t, h):
            g = gif_ref[pl.ds(t, 1), :] + gib_ref[pl.ds(N - 1 - t, 1), :]
            gh = jnp.dot(h.astype(jnp.bfloat16), w,
                         preferred_element_type=_f32)
            rg = 0.5 + 0.5 * jnp.tanh(0.5 * (g[:, 0:128] + gh[:, 0:128]))
            zg = 0.5 + 0.5 * jnp.tanh(0.5 * (g[:, 128:256] + gh[:, 128:256]))
            ng = jnp.tanh(g[:, 256:384] + rg * (gh[:, 256:384] + bnv))
            h2 = ng + zg * (h - ng)
            f_ref[pl.ds(t, 1), :] = h2
            r_ref[pl.ds(N - 1 - t, 1), :] = h2
            return h2

        def step4(i, h):
            h = step(4 * i + 1, step(4 * i, h))
            return step(4 * i + 3, step(4 * i + 2, h))

        lax.fori_loop(0, N // 4, step4, jnp.zeros((1, 128), _f32))

    return pl.pallas_call(
        body,
        out_shape=[jax.ShapeDtypeStruct((N, 128), _f32),
                   jax.ShapeDtypeStruct((N, 128), _f32)],
    )(gif, gib, wblk.astype(jnp.bfloat16), bn)


def _final(f1, r1, wlt, wlb, blr):
    def body(fr, rr, wt, wb, bref, out):
        out[...] = (jnp.dot(fr[...], wt[...], preferred_element_type=_f32)
                    + jnp.dot(rr[...], wb[...], preferred_element_type=_f32)
                    + bref[...])

    return pl.pallas_call(
        body,
        grid=(_G,),
        in_specs=[_row_spec(128), _row_spec(128),
                  _full_spec(128, 10), _full_spec(128, 10), _full_spec(1, 10)],
        out_specs=_row_spec(10),
        out_shape=jax.ShapeDtypeStruct((N, 10), _f32),
    )(f1, r1, wlt, wlb, blr)


def _pack_gi(Wih, bih, bhh, off):
    T = Wih.T.astype(_f32)
    inn = T.shape[0]
    w = jnp.zeros((inn, 384), _f32)
    w = w.at[:, off:off + 64].set(T[:, 0:64])
    w = w.at[:, 128 + off:128 + off + 64].set(T[:, 64:128])
    w = w.at[:, 256 + off:256 + off + 64].set(T[:, 128:192])
    cv = jnp.zeros((384,), _f32)
    cv = cv.at[off:off + 64].set(bih[0:64] + bhh[0:64])
    cv = cv.at[128 + off:128 + off + 64].set(bih[64:128] + bhh[64:128])
    cv = cv.at[256 + off:256 + off + 64].set(bih[128:192])
    return w, cv.reshape(1, 384)


def _pack_whh(Whh_f, Whh_b, bhh_f, bhh_b):
    w = jnp.zeros((128, 384), _f32)
    w = w.at[0:64, 0:64].set(Whh_f[0:64].T)
    w = w.at[64:128, 64:128].set(Whh_b[0:64].T)
    w = w.at[0:64, 128:192].set(Whh_f[64:128].T)
    w = w.at[64:128, 192:256].set(Whh_b[64:128].T)
    w = w.at[0:64, 256:320].set(Whh_f[128:192].T)
    w = w.at[64:128, 320:384].set(Whh_b[128:192].T)
    bn = jnp.concatenate([bhh_f[128:192], bhh_b[128:192]]).reshape(1, 128)
    return w, bn


def kernel(x, edge_index, W1, b1, W2, b2,
           l0f_Wih, l0f_Whh, l0f_bih, l0f_bhh,
           l0b_Wih, l0b_Whh, l0b_bih, l0b_bhh,
           l1f_Wih, l1f_Whh, l1f_bih, l1f_bhh,
           l1b_Wih, l1b_Whh, l1b_bih, l1b_bhh, Wl, bl):
    pad = EPAD - E
    src2 = jnp.concatenate(
        [edge_index[0], jnp.zeros((pad,), jnp.int32)]).reshape(EPAD // CW, CW)
    dst2 = jnp.concatenate(
        [edge_index[1], jnp.full((pad,), N, jnp.int32)]).reshape(EPAD // CW, CW)

    degp = _sc_degree(dst2)
    h1p, dinv = _prep1(degp[0, :N, 0:1], degp[1, :N, 0:1], x, W1)
    acc1 = _sc_scatter(h1p, src2, dst2, 16)
    h2p = _mid(acc1[0, :N], acc1[1, :N], h1p, dinv, b1.reshape(1, 16), W2)
    acc2 = _sc_scatter(h2p, src2, dst2, 32)

    w0f, c0f = _pack_gi(l0f_Wih, l0f_bih, l0f_bhh, 0)
    w0b, c0b = _pack_gi(l0b_Wih, l0b_bih, l0b_bhh, 64)
    gi0f, gi0b = _gi0(acc2[0, :N], acc2[1, :N], h2p, dinv,
                      b2.reshape(1, 32), w0f, c0f, w0b, c0b)
    wblk0, bn0 = _pack_whh(l0f_Whh, l0b_Whh, l0f_bhh, l0b_bhh)
    f0, r0 = _gru(gi0f, gi0b, wblk0, bn0)

    z64 = jnp.zeros((64, 384), _f32)
    w1f, c1f = _pack_gi(l1f_Wih, l1f_bih, l1f_bhh, 0)
    w1b, c1b = _pack_gi(l1b_Wih, l1b_bih, l1b_bhh, 64)
    gi1f, gi1b = _gi1(f0, r0,
                      jnp.concatenate([w1f[0:64], z64]),
                      jnp.concatenate([z64, w1f[64:128]]), c1f,
                      jnp.concatenate([w1b[0:64], z64]),
                      jnp.concatenate([z64, w1b[64:128]]), c1b)
    wblk1, bn1 = _pack_whh(l1f_Whh, l1b_Whh, l1f_bhh, l1b_bhh)
    f1, r1 = _gru(gi1f, gi1b, wblk1, bn1)

    zl = jnp.zeros((64, 10), _f32)
    return _final(f1, r1,
                  jnp.concatenate([Wl[0:64], zl]),
                  jnp.concatenate([zl, Wl[64:128]]), bl.reshape(1, 10))


# speedup vs baseline: 15.7791x; 1.0033x over previous
import functools

import jax
import jax.numpy as jnp
from jax import lax
from jax.experimental import pallas as pl
from jax.experimental.pallas import tpu as pltpu
import jax.experimental.pallas.tpu_sc as plsc

N = 10000
E = 320000
NP = 10240
SROWS = NP // 16
CW = 128
NWORK = 32
KCH = 80
EPAD = NWORK * KCH * CW
GH = 64

_f32 = jnp.float32


def _mesh():
    return plsc.VectorSubcoreMesh(core_axis_name="c", subcore_axis_name="s")


def _sc_degree(dst2):
    @functools.partial(
        pl.kernel,
        out_type=jax.ShapeDtypeStruct((2, NP, 16), _f32),
        mesh=_mesh(),
        compiler_params=pltpu.CompilerParams(use_tc_tiling_on_sc=False),
        scratch_types=[
            pltpu.VMEM((KCH, CW), jnp.int32),
            pltpu.VMEM((CW, 16), _f32),
            pltpu.VMEM_SHARED((NP, 16), _f32),
        ],
    )
    def k(dst_hbm, ones_hbm, zeros_hbm, out_hbm, dst_v, ones_v, acc_sh):
        c = lax.axis_index("c")
        s = lax.axis_index("s")
        w = s * 2 + c
        pltpu.sync_copy(zeros_hbm.at[pl.ds(s * SROWS, SROWS)],
                        acc_sh.at[pl.ds(s * SROWS, SROWS)])
        pltpu.sync_copy(ones_hbm, ones_v)
        pltpu.sync_copy(dst_hbm.at[pl.ds(w * KCH, KCH)], dst_v)
        plsc.subcore_barrier()

        def body(j, carry):
            pltpu.sync_copy(ones_v, acc_sh.at[dst_v.at[j]], add=True)
            return carry

        lax.fori_loop(0, KCH, body, 0)
        plsc.subcore_barrier()
        pltpu.sync_copy(acc_sh.at[pl.ds(s * SROWS, SROWS)],
                        out_hbm.at[c, pl.ds(s * SROWS, SROWS)])

    return k(dst2, jnp.ones((CW, 16), _f32), jnp.zeros((NP, 16), _f32))


def _sc_scatter(hp, src2, dst2, width):
    @functools.partial(
        pl.kernel,
        out_type=jax.ShapeDtypeStruct((2, NP, width), _f32),
        mesh=_mesh(),
        compiler_params=pltpu.CompilerParams(use_tc_tiling_on_sc=False),
        scratch_types=[
            pltpu.VMEM((KCH, CW), jnp.int32),
            pltpu.VMEM((KCH, CW), jnp.int32),
            pltpu.VMEM((CW, width), _f32),
            pltpu.VMEM((CW, width), _f32),
            pltpu.VMEM_SHARED((NP, width), _f32),
            pltpu.SemaphoreType.DMA,
            pltpu.SemaphoreType.DMA,
        ],
    )
    def k(hp_hbm, src_hbm, dst_hbm, zeros_hbm, out_hbm,
          src_v, dst_v, rows_a, rows_b, acc_sh, sem_a, sem_b):
        c = lax.axis_index("c")
        s = lax.axis_index("s")
        w = s * 2 + c
        pltpu.sync_copy(zeros_hbm.at[pl.ds(s * SROWS, SROWS)],
                        acc_sh.at[pl.ds(s * SROWS, SROWS)])
        pltpu.sync_copy(src_hbm.at[pl.ds(w * KCH, KCH)], src_v)
        pltpu.sync_copy(dst_hbm.at[pl.ds(w * KCH, KCH)], dst_v)
        plsc.subcore_barrier()

        pltpu.async_copy(hp_hbm.at[src_v.at[0]], rows_a, sem_a)

        def body(j, carry):
            @pl.when(j % 2 == 0)
            def _():
                pltpu.make_async_copy(zeros_hbm.at[pl.ds(0, CW)], rows_a,
                                      sem_a).wait()
                @pl.when(j + 1 < KCH)
                def _():
                    pltpu.async_copy(hp_hbm.at[src_v.at[j + 1]], rows_b,
                                     sem_b)
                pltpu.sync_copy(rows_a, acc_sh.at[dst_v.at[j]], add=True)

            @pl.when(j % 2 == 1)
            def _():
                pltpu.make_async_copy(zeros_hbm.at[pl.ds(0, CW)], rows_b,
                                      sem_b).wait()
                @pl.when(j + 1 < KCH)
                def _():
                    pltpu.async_copy(hp_hbm.at[src_v.at[j + 1]], rows_a,
                                     sem_a)
                pltpu.sync_copy(rows_b, acc_sh.at[dst_v.at[j]], add=True)

            return carry

        lax.fori_loop(0, KCH, body, 0)
        plsc.subcore_barrier()
        pltpu.sync_copy(acc_sh.at[pl.ds(s * SROWS, SROWS)],
                        out_hbm.at[c, pl.ds(s * SROWS, SROWS)])

    return k(hp, src2, dst2, jnp.zeros((NP, width), _f32))


_R = 1000
_G = N // _R


def _row_spec(width):
    return pl.BlockSpec((_R, width), lambda i: (i, 0))


def _full_spec(a, b):
    return pl.BlockSpec((a, b), lambda i: (0, 0))


def _prep1(dega, degb, x, W1):
    def body(da, db, xr, w1, h1p, dv):
        dinv = lax.rsqrt(da[...] + db[...] + 1.0)
        h1p[...] = jnp.dot(xr[...], w1[...],
                           preferred_element_type=_f32) * dinv
        dv[...] = dinv

    return pl.pallas_call(
        body,
        grid=(_G,),
        in_specs=[_row_spec(1), _row_spec(1), _row_spec(128),
                  _full_spec(128, 16)],
        out_specs=[_row_spec(16), _row_spec(1)],
        out_shape=[jax.ShapeDtypeStruct((N, 16), _f32),
                   jax.ShapeDtypeStruct((N, 1), _f32)],
    )(dega, degb, x, W1)


def _mid(a0, a1, h1p, dinv, b1r, W2):
    def body(a0r, a1r, hpr, dvr, b1ref, w2, out):
        h1 = jnp.maximum((a0r[...] + a1r[...] + hpr[...]) * dvr[...]
                         + b1ref[...], 0.0)
        out[...] = jnp.dot(h1, w2[...], preferred_element_type=_f32) * dvr[...]

    return pl.pallas_call(
        body,
        grid=(_G,),
        in_specs=[_row_spec(16), _row_spec(16), _row_spec(16), _row_spec(1),
                  _full_spec(1, 16), _full_spec(16, 32)],
        out_specs=_row_spec(32),
        out_shape=jax.ShapeDtypeStruct((N, 32), _f32),
    )(a0, a1, h1p, dinv, b1r, W2)


def _gi0(a0, a1, h2p, dinv, b2r, w0f, c0f, w0b, c0b):
    def body(a0r, a1r, hpr, dvr, b2ref, wf, cf, wb, cb, of, ob):
        h2 = jnp.maximum((a0r[...] + a1r[...] + hpr[...]) * dvr[...]
                         + b2ref[...], 0.0)
        of[...] = jnp.dot(h2, wf[...], preferred_element_type=_f32) + cf[...]
        ob[...] = jnp.dot(h2, wb[...], preferred_element_type=_f32) + cb[...]

    return pl.pallas_call(
        body,
        grid=(_G,),
        in_specs=[_row_spec(32), _row_spec(32), _row_spec(32), _row_spec(1),
                  _full_spec(1, 32), _full_spec(32, 384), _full_spec(1, 384),
                  _full_spec(32, 384), _full_spec(1, 384)],
        out_specs=[_row_spec(384), _row_spec(384)],
        out_shape=[jax.ShapeDtypeStruct((N, 384), _f32),
                   jax.ShapeDtypeStruct((N, 384), _f32)],
    )(a0, a1, h2p, dinv, b2r, w0f, c0f, w0b, c0b)


def _gi1(f0, r0, wft, wfb, c1f, wbt, wbb, c1b):
    def body(fr, rr, wft_, wfb_, cf, wbt_, wbb_, cb, of, ob):
        f = fr[...]
        r = rr[...]
        of[...] = (jnp.dot(f, wft_[...], preferred_element_type=_f32)
                   + jnp.dot(r, wfb_[...], preferred_element_type=_f32)
                   + cf[...])
        ob[...] = (jnp.dot(f, wbt_[...], preferred_element_type=_f32)
                   + jnp.dot(r, wbb_[...], preferred_element_type=_f32)
                   + cb[...])

    return pl.pallas_call(
        body,
        grid=(_G,),
        in_specs=[_row_spec(128), _row_spec(128),
                  _full_spec(128, 384), _full_spec(128, 384),
                  _full_spec(1, 384),
                  _full_spec(128, 384), _full_spec(128, 384),
                  _full_spec(1, 384)],
        out_specs=[_row_spec(384), _row_spec(384)],
        out_shape=[jax.ShapeDtypeStruct((N, 384), _f32),
                   jax.ShapeDtypeStruct((N, 384), _f32)],
    )(f0, r0, wft, wfb, c1f, wbt, wbb, c1b)


def _gru(gif, gib, wblk, bn, fin=None):
    def body(gif_ref, gib_ref, wblk_ref, bn_ref, *rest):
        if fin is None:
            f_ref, r_ref = rest
        else:
            wlt_ref, wlb_ref, bl_ref, f_ref, r_ref, o_ref = rest
        w = wblk_ref[...]
        bnv = bn_ref[...]

        def step(t, h):
            g = gif_ref[pl.ds(t, 1), :] + gib_ref[pl.ds(N - 1 - t, 1), :]
            gh = jnp.dot(h.astype(jnp.bfloat16), w,
                         preferred_element_type=_f32)
            rg = 0.5 + 0.5 * jnp.tanh(0.5 * (g[:, 0:128] + gh[:, 0:128]))
            zg = 0.5 + 0.5 * jnp.tanh(0.5 * (g[:, 128:256] + gh[:, 128:256]))
            ng = jnp.tanh(g[:, 256:384] + rg * (gh[:, 256:384] + bnv))
            h2 = ng + zg * (h - ng)
            f_ref[pl.ds(t, 1), :] = h2
            r_ref[pl.ds(N - 1 - t, 1), :] = h2
            return h2

        def step4(i, h):
            h = step(4 * i + 1, step(4 * i, h))
            return step(4 * i + 3, step(4 * i + 2, h))

        lax.fori_loop(0, N // 4, step4, jnp.zeros((1, 128), _f32))

        if fin is not None:
            o_ref[...] = (jnp.dot(f_ref[...], wlt_ref[...],
                                  preferred_element_type=_f32)
                          + jnp.dot(r_ref[...], wlb_ref[...],
                                    preferred_element_type=_f32)
                          + bl_ref[...])

    shapes = [jax.ShapeDtypeStruct((N, 128), _f32),
              jax.ShapeDtypeStruct((N, 128), _f32)]
    args = (gif, gib, wblk.astype(jnp.bfloat16), bn)
    if fin is not None:
        shapes.append(jax.ShapeDtypeStruct((N, 10), _f32))
        args = args + tuple(fin)
    out = pl.pallas_call(body, out_shape=shapes)(*args)
    return out[2] if fin is not None else (out[0], out[1])


def _final(f1, r1, wlt, wlb, blr):
    def body(fr, rr, wt, wb, bref, out):
        out[...] = (jnp.dot(fr[...], wt[...], preferred_element_type=_f32)
                    + jnp.dot(rr[...], wb[...], preferred_element_type=_f32)
                    + bref[...])

    return pl.pallas_call(
        body,
        grid=(_G,),
        in_specs=[_row_spec(128), _row_spec(128),
                  _full_spec(128, 10), _full_spec(128, 10), _full_spec(1, 10)],
        out_specs=_row_spec(10),
        out_shape=jax.ShapeDtypeStruct((N, 10), _f32),
    )(f1, r1, wlt, wlb, blr)


def _pack_gi(Wih, bih, bhh, off):
    T = Wih.T.astype(_f32)
    inn = T.shape[0]
    w = jnp.zeros((inn, 384), _f32)
    w = w.at[:, off:off + 64].set(T[:, 0:64])
    w = w.at[:, 128 + off:128 + off + 64].set(T[:, 64:128])
    w = w.at[:, 256 + off:256 + off + 64].set(T[:, 128:192])
    cv = jnp.zeros((384,), _f32)
    cv = cv.at[off:off + 64].set(bih[0:64] + bhh[0:64])
    cv = cv.at[128 + off:128 + off + 64].set(bih[64:128] + bhh[64:128])
    cv = cv.at[256 + off:256 + off + 64].set(bih[128:192])
    return w, cv.reshape(1, 384)


def _pack_whh(Whh_f, Whh_b, bhh_f, bhh_b):
    w = jnp.zeros((128, 384), _f32)
    w = w.at[0:64, 0:64].set(Whh_f[0:64].T)
    w = w.at[64:128, 64:128].set(Whh_b[0:64].T)
    w = w.at[0:64, 128:192].set(Whh_f[64:128].T)
    w = w.at[64:128, 192:256].set(Whh_b[64:128].T)
    w = w.at[0:64, 256:320].set(Whh_f[128:192].T)
    w = w.at[64:128, 320:384].set(Whh_b[128:192].T)
    bn = jnp.concatenate([bhh_f[128:192], bhh_b[128:192]]).reshape(1, 128)
    return w, bn


def kernel(x, edge_index, W1, b1, W2, b2,
           l0f_Wih, l0f_Whh, l0f_bih, l0f_bhh,
           l0b_Wih, l0b_Whh, l0b_bih, l0b_bhh,
           l1f_Wih, l1f_Whh, l1f_bih, l1f_bhh,
           l1b_Wih, l1b_Whh, l1b_bih, l1b_bhh, Wl, bl):
    pad = EPAD - E
    src2 = jnp.concatenate(
        [edge_index[0], jnp.zeros((pad,), jnp.int32)]).reshape(EPAD // CW, CW)
    dst2 = jnp.concatenate(
        [edge_index[1], jnp.full((pad,), N, jnp.int32)]).reshape(EPAD // CW, CW)

    degp = _sc_degree(dst2)
    h1p, dinv = _prep1(degp[0, :N, 0:1], degp[1, :N, 0:1], x, W1)
    acc1 = _sc_scatter(h1p, src2, dst2, 16)
    h2p = _mid(acc1[0, :N], acc1[1, :N], h1p, dinv, b1.reshape(1, 16), W2)
    acc2 = _sc_scatter(h2p, src2, dst2, 32)

    w0f, c0f = _pack_gi(l0f_Wih, l0f_bih, l0f_bhh, 0)
    w0b, c0b = _pack_gi(l0b_Wih, l0b_bih, l0b_bhh, 64)
    gi0f, gi0b = _gi0(acc2[0, :N], acc2[1, :N], h2p, dinv,
                      b2.reshape(1, 32), w0f, c0f, w0b, c0b)
    wblk0, bn0 = _pack_whh(l0f_Whh, l0b_Whh, l0f_bhh, l0b_bhh)
    f0, r0 = _gru(gi0f, gi0b, wblk0, bn0)

    z64 = jnp.zeros((64, 384), _f32)
    w1f, c1f = _pack_gi(l1f_Wih, l1f_bih, l1f_bhh, 0)
    w1b, c1b = _pack_gi(l1b_Wih, l1b_bih, l1b_bhh, 64)
    gi1f, gi1b = _gi1(f0, r0,
                      jnp.concatenate([w1f[0:64], z64]),
                      jnp.concatenate([z64, w1f[64:128]]), c1f,
                      jnp.concatenate([w1b[0:64], z64]),
                      jnp.concatenate([z64, w1b[64:128]]), c1b)
    wblk1, bn1 = _pack_whh(l1f_Whh, l1b_Whh, l1f_bhh, l1b_bhh)
    zl = jnp.zeros((64, 10), _f32)
    return _gru(gi1f, gi1b, wblk1, bn1,
                fin=(jnp.concatenate([Wl[0:64], zl]),
                     jnp.concatenate([zl, Wl[64:128]]),
                     bl.reshape(1, 10)))
